# baseline (speedup 1.0000x reference)
import functools

import jax
import jax.numpy as jnp
from jax import lax
from jax.experimental import pallas as pl
from jax.experimental.pallas import tpu as pltpu
from jax.experimental.pallas import tpu_sc as plsc

N = 10000
E = 320000
H = 128
HH = H // 2
NC = 2
NS = 16
K = 256
EP = 327680
NROW = 10240
RPT = NROW // NS

_f32 = jnp.float32


def _mesh():
    return plsc.VectorSubcoreMesh(core_axis_name="c", subcore_axis_name="s")


_SC_PARAMS = pltpu.CompilerParams(use_tc_tiling_on_sc=False)


def _zero_vmem2d(buf, rows, width):
    @pl.loop(0, rows)
    def _(i):
        @pl.loop(0, width, step=16)
        def _(j):
            buf[i, pl.ds(j, 16)] = jnp.zeros((16,), _f32)


def _fill_ones2d(buf, rows, width):
    @pl.loop(0, rows)
    def _(i):
        @pl.loop(0, width, step=16)
        def _(j):
            buf[i, pl.ds(j, 16)] = jnp.ones((16,), _f32)


def _zero_shared_rows(zbuf, zrows, table, row0, nrows):
    full, rem = nrows // zrows, nrows % zrows
    for q in range(full):
        pltpu.sync_copy(zbuf, table.at[pl.ds(row0 + q * zrows, zrows)])
    if rem:
        pltpu.sync_copy(zbuf.at[pl.ds(0, rem)],
                        table.at[pl.ds(row0 + full * zrows, rem)])


def _sc_degree(dst2):
    nchunk = EP // (K * NC * NS)

    @functools.partial(
        pl.kernel,
        out_type=jax.ShapeDtypeStruct((NC, NROW, 16), _f32),
        mesh=_mesh(),
        compiler_params=_SC_PARAMS,
        scratch_types=[
            pltpu.VMEM((nchunk, K), jnp.int32),
            pltpu.VMEM((K, 16), _f32),
            pltpu.VMEM((80, 16), _f32),
            pltpu.VMEM((160, 16), _f32),
            pltpu.VMEM_SHARED((NROW, 16), _f32),
            pltpu.SemaphoreType.DMA,
        ],
    )
    def k(dst_hbm, out_hbm, dst2_v, ones_v, zb_v, stage_v, table_sh, sem):
        c = lax.axis_index("c")
        s = lax.axis_index("s")
        wid = c * NS + s
        _fill_ones2d(ones_v, K, 16)
        _zero_vmem2d(zb_v, 80, 16)
        _zero_shared_rows(zb_v, 80, table_sh, s * RPT, RPT)
        pltpu.sync_copy(dst_hbm.at[pl.ds(wid * nchunk, nchunk)], dst2_v)
        plsc.subcore_barrier()

        @pl.loop(0, nchunk)
        def _(j):
            pltpu.async_copy(ones_v, table_sh.at[dst2_v.at[j]], sem,
                             add=True)

        @pl.loop(0, nchunk)
        def _(j):
            pltpu.make_async_copy(ones_v, table_sh.at[dst2_v.at[0]],
                                  sem).wait()

        plsc.subcore_barrier()
        for half in range(4):
            r0 = s * RPT + half * 160
            pltpu.sync_copy(table_sh.at[pl.ds(r0, 160)], stage_v)
            pltpu.sync_copy(stage_v, out_hbm.at[c, pl.ds(r0, 160)])

    return k(dst2)


def _add_offset(idx2_v, nrows, off):
    @pl.loop(0, nrows)
    def _(i):
        @pl.loop(0, K, step=16)
        def _(j):
            idx2_v[i, pl.ds(j, 16)] = idx2_v[i, pl.ds(j, 16)] + off


def _pipe_gather_scatter(nchunk, streams):
    def start_g(j, b):
        for (src_hbm, idx2_v, _, bufs, gsems, _ss, _t) in streams:
            pltpu.async_copy(src_hbm.at[idx2_v.at[j]], bufs[b], gsems[b])

    def wait_g(b):
        for (src_hbm, idx2_v, _, bufs, gsems, _ss, _t) in streams:
            pltpu.make_async_copy(src_hbm.at[idx2_v.at[0]], bufs[b],
                                  gsems[b]).wait()

    def scat(j, b):
        for (_s, _i, dst2_v, bufs, _gs, _ss, table_sh) in streams:
            pltpu.sync_copy(bufs[b], table_sh.at[dst2_v.at[j]], add=True)

    start_g(0, 0)
    start_g(1, 1)

    @pl.loop(0, nchunk, step=2)
    def _(j):
        wait_g(0)
        scat(j, 0)

        @pl.when(j + 2 < nchunk)
        def _():
            start_g(j + 2, 0)

        @pl.when(j + 1 < nchunk)
        def _():
            wait_g(1)
            scat(j + 1, 1)

        @pl.when(j + 3 < nchunk)
        def _():
            start_g(j + 3, 1)


def _sc_gcn_agg(xwp4, src2, dst2):
    nchunk = EP // (K * NS)

    @functools.partial(
        pl.kernel,
        out_type=jax.ShapeDtypeStruct((NC, 2, NROW, HH), _f32),
        mesh=_mesh(),
        compiler_params=_SC_PARAMS,
        scratch_types=[
            pltpu.VMEM((nchunk, K), jnp.int32),
            pltpu.VMEM((nchunk, K), jnp.int32),
            pltpu.VMEM((K, HH), _f32),
            pltpu.VMEM((K, HH), _f32),
            pltpu.VMEM((160, HH), _f32),
            pltpu.VMEM_SHARED((NROW, HH), _f32),
            pltpu.SemaphoreType.DMA,
            pltpu.SemaphoreType.DMA,
            pltpu.SemaphoreType.DMA,
            pltpu.SemaphoreType.DMA,
        ],
    )
    def k(x_hbm, src_hbm, dst_hbm, out_hbm, src2_v, dst2_v, buf0, buf1,
          stage_v, table_sh, sem0, sem1, sem2, sem3):
        c = lax.axis_index("c")
        s = lax.axis_index("s")
        pltpu.sync_copy(src_hbm.at[pl.ds(s * nchunk, nchunk)], src2_v)
        pltpu.sync_copy(dst_hbm.at[pl.ds(s * nchunk, nchunk)], dst2_v)
        _add_offset(src2_v, nchunk, c * (2 * N))
        for q in range(2):
            if q == 1:
                _add_offset(src2_v, nchunk, N)
            _zero_vmem2d(buf0, K, HH)
            _zero_shared_rows(buf0, K, table_sh, s * RPT, RPT)
            plsc.subcore_barrier()
            _pipe_gather_scatter(
                nchunk,
                [(x_hbm, src2_v, dst2_v, (buf0, buf1), (sem0, sem1),
                  (sem2, sem3), table_sh)])
            plsc.subcore_barrier()
            for half in range(4):
                r0 = s * RPT + half * 160
                pltpu.sync_copy(table_sh.at[pl.ds(r0, 160)], stage_v)
                pltpu.sync_copy(stage_v, out_hbm.at[c, q, pl.ds(r0, 160)])

    return k(xwp4, src2, dst2)


def _sc_attn_rows(m2, src2, dst2):
    nchunk = EP // (K * NC * NS)

    @functools.partial(
        pl.kernel,
        out_type=jax.ShapeDtypeStruct((NC, 2, NROW, HH), _f32),
        mesh=_mesh(),
        compiler_params=_SC_PARAMS,
        scratch_types=[
            pltpu.VMEM((nchunk, K), jnp.int32),
            pltpu.VMEM((nchunk, K), jnp.int32),
            pltpu.VMEM((K, HH), _f32),
            pltpu.VMEM((K, HH), _f32),
            pltpu.VMEM((160, HH), _f32),
            pltpu.VMEM_SHARED((NROW, HH), _f32),
            pltpu.SemaphoreType.DMA,
            pltpu.SemaphoreType.DMA,
            pltpu.SemaphoreType.DMA,
            pltpu.SemaphoreType.DMA,
        ],
    )
    def k(m_hbm, src_hbm, dst_hbm, out_hbm, src2_v, dst2_v, buf0, buf1,
          stage_v, table_sh, sem0, sem1, sem2, sem3):
        c = lax.axis_index("c")
        s = lax.axis_index("s")
        wid = c * NS + s
        pltpu.sync_copy(src_hbm.at[pl.ds(wid * nchunk, nchunk)], src2_v)
        pltpu.sync_copy(dst_hbm.at[pl.ds(wid * nchunk, nchunk)], dst2_v)
        for q in range(2):
            if q == 1:
                _add_offset(src2_v, nchunk, N)
            _zero_vmem2d(buf0, K, HH)
            _zero_shared_rows(buf0, K, table_sh, s * RPT, RPT)
            plsc.subcore_barrier()
            _pipe_gather_scatter(
                nchunk,
                [(m_hbm, src2_v, dst2_v, (buf0, buf1), (sem0, sem1),
                  (sem2, sem3), table_sh)])
            plsc.subcore_barrier()
            for half in range(4):
                r0 = s * RPT + half * 160
                pltpu.sync_copy(table_sh.at[pl.ds(r0, 160)], stage_v)
                pltpu.sync_copy(stage_v, out_hbm.at[c, q, pl.ds(r0, 160)])

    return k(m2, src2, dst2)


def _sc_attn_denom(ms_rep, src2, dst2):
    nchunk = EP // (K * NC * NS)

    @functools.partial(
        pl.kernel,
        out_type=jax.ShapeDtypeStruct((NC, NROW, 16), _f32),
        mesh=_mesh(),
        compiler_params=_SC_PARAMS,
        scratch_types=[
            pltpu.VMEM((nchunk, K), jnp.int32),
            pltpu.VMEM((nchunk, K), jnp.int32),
            pltpu.VMEM((K, 16), _f32),
            pltpu.VMEM((K, 16), _f32),
            pltpu.VMEM((160, 16), _f32),
            pltpu.VMEM_SHARED((NROW, 16), _f32),
            pltpu.SemaphoreType.DMA,
            pltpu.SemaphoreType.DMA,
            pltpu.SemaphoreType.DMA,
            pltpu.SemaphoreType.DMA,
        ],
    )
    def k(msr_hbm, src_hbm, dst_hbm, out_hbm, src2_v, dst2_v, buf0, buf1,
          stage_v, table_sh, sem0, sem1, sem2, sem3):
        c = lax.axis_index("c")
        s = lax.axis_index("s")
        wid = c * NS + s
        pltpu.sync_copy(src_hbm.at[pl.ds(wid * nchunk, nchunk)], src2_v)
        pltpu.sync_copy(dst_hbm.at[pl.ds(wid * nchunk, nchunk)], dst2_v)
        _zero_vmem2d(buf0, K, 16)
        _zero_shared_rows(buf0, K, table_sh, s * RPT, RPT)
        plsc.subcore_barrier()
        _pipe_gather_scatter(
            nchunk,
            [(msr_hbm, src2_v, dst2_v, (buf0, buf1), (sem0, sem1),
              (sem2, sem3), table_sh)])
        plsc.subcore_barrier()
        for half in range(4):
            r0 = s * RPT + half * 160
            pltpu.sync_copy(table_sh.at[pl.ds(r0, 160)], stage_v)
            pltpu.sync_copy(stage_v, out_hbm.at[c, pl.ds(r0, 160)])

    return k(ms_rep, src2, dst2)


BN = 1000
_GRID = N // BN
_HI = lax.Precision.HIGHEST


def _dot(a, b):
    return jnp.dot(a, b, preferred_element_type=_f32, precision=_HI)


def _lrelu_(v):
    return jnp.where(v > 0, v, 0.2 * v)


def _mlp3_(v, W1, b1, W2, b2, W3, b3):
    v = _lrelu_(_dot(v, W1) + b1)
    v = _lrelu_(_dot(v, W2) + b2)
    return _dot(v, W3) + b3


def _tc_xwp(x, Wg2, deg2):
    def body(x_ref, w_ref, deg_ref, o_ref):
        q = (pl.program_id(0) // _GRID) % 2
        deg = deg_ref[0, 0, :, 0] + deg_ref[0, 1, :, 0] + 1.0
        dinv = lax.rsqrt(deg)[:, None]
        xw = _dot(x_ref[...], w_ref[0])
        half = jnp.where(q == 0, xw[:, :HH], xw[:, HH:])
        o_ref[...] = half * dinv

    return pl.pallas_call(
        body,
        grid=(4 * _GRID,),
        in_specs=[
            pl.BlockSpec((BN, H), lambda i: (i % _GRID, 0)),
            pl.BlockSpec((1, H, H), lambda i: (i // (2 * _GRID), 0, 0)),
            pl.BlockSpec((1, NC, BN, 16), lambda i: (0, 0, i % _GRID, 0)),
        ],
        out_specs=pl.BlockSpec((BN, HH), lambda i: (i, 0)),
        out_shape=jax.ShapeDtypeStruct((4 * N, HH), _f32),
    )(x, Wg2, deg2[None])


def _tc_gcn_post(agg4, xwp4, deg2, bgo, bgt, Wa, ba):
    def body(agg_ref, xo0_ref, xo1_ref, xt0_ref, xt1_ref, deg_ref,
             bgo_ref, bgt_ref, wao_ref, wat_ref, ba_ref,
             ro_ref, rt_ref, sc_ref, g_ref):
        i = pl.program_id(0)
        deg = deg_ref[0, 0, :, 0] + deg_ref[0, 1, :, 0] + 1.0
        dinv = lax.rsqrt(deg)[:, None]
        agg_o = jnp.concatenate([agg_ref[0, 0, 0], agg_ref[0, 0, 1]], axis=1)
        agg_t = jnp.concatenate([agg_ref[0, 1, 0], agg_ref[0, 1, 1]], axis=1)
        xwp_o = jnp.concatenate([xo0_ref[...], xo1_ref[...]], axis=1)
        xwp_t = jnp.concatenate([xt0_ref[...], xt1_ref[...]], axis=1)
        ro = jnp.maximum((agg_o + xwp_o) * dinv + bgo_ref[...], 0.0)
        rt = jnp.maximum((agg_t + xwp_t) * dinv + bgt_ref[...], 0.0)
        sc = _lrelu_(_dot(ro, wao_ref[...]) + _dot(rt, wat_ref[...])
                     + ba_ref[...])
        ro_ref[...] = ro
        rt_ref[...] = rt
        sc_ref[...] = sc

        @pl.when(i == 0)
        def _():
            g_ref[...] = jnp.full((1, 1), -jnp.inf, _f32)

        g_ref[...] = jnp.maximum(g_ref[...], jnp.max(sc))

    hspec = pl.BlockSpec((BN, HH), lambda i: (i, 0))
    return pl.pallas_call(
        body,
        grid=(_GRID,),
        in_specs=[
            pl.BlockSpec((1, NC, 2, BN, HH), lambda i: (0, 0, 0, i, 0)),
            pl.BlockSpec((BN, HH), lambda i: (i, 0)),
            pl.BlockSpec((BN, HH), lambda i: (_GRID + i, 0)),
            pl.BlockSpec((BN, HH), lambda i: (2 * _GRID + i, 0)),
            pl.BlockSpec((BN, HH), lambda i: (3 * _GRID + i, 0)),
            pl.BlockSpec((1, NC, BN, 16), lambda i: (0, 0, i, 0)),
            pl.BlockSpec((1, H), lambda i: (0, 0)),
            pl.BlockSpec((1, H), lambda i: (0, 0)),
            pl.BlockSpec((H, 1), lambda i: (0, 0)),
            pl.BlockSpec((H, 1), lambda i: (0, 0)),
            pl.BlockSpec((1, 1), lambda i: (0, 0)),
        ],
        out_specs=[
            pl.BlockSpec((BN, H), lambda i: (i, 0)),
            pl.BlockSpec((BN, H), lambda i: (i, 0)),
            pl.BlockSpec((BN, 1), lambda i: (i, 0)),
            pl.BlockSpec((1, 1), lambda i: (0, 0)),
        ],
        out_shape=[
            jax.ShapeDtypeStruct((N, H), _f32),
            jax.ShapeDtypeStruct((N, H), _f32),
            jax.ShapeDtypeStruct((N, 1), _f32),
            jax.ShapeDtypeStruct((1, 1), _f32),
        ],
    )(agg4[None], xwp4, xwp4, xwp4, xwp4, deg2[None], bgo, bgt,
      Wa[:H], Wa[H:], ba)


def _tc_msg(score, g, t2, r_t):
    def body(sc_ref, g_ref, rt_ref, t_ref, m_ref, msr_ref):
        q = pl.program_id(0) // _GRID
        ms = jnp.where(t_ref[...] > 0,
                       jnp.exp(sc_ref[...] - g_ref[...]), 0.0)
        rt = rt_ref[...]
        half = jnp.where(q == 0, rt[:, :HH], rt[:, HH:])
        m_ref[...] = ms * half
        msr_ref[...] = jnp.broadcast_to(ms, (BN, 16))

    return pl.pallas_call(
        body,
        grid=(2 * _GRID,),
        in_specs=[
            pl.BlockSpec((BN, 1), lambda i: (i % _GRID, 0)),
            pl.BlockSpec((1, 1), lambda i: (0, 0)),
            pl.BlockSpec((BN, H), lambda i: (i % _GRID, 0)),
            pl.BlockSpec((BN, 1), lambda i: (i % _GRID, 0)),
        ],
        out_specs=[
            pl.BlockSpec((BN, HH), lambda i: (i, 0)),
            pl.BlockSpec((BN, 16), lambda i: (i % _GRID, 0)),
        ],
        out_shape=[
            jax.ShapeDtypeStruct((2 * N, HH), _f32),
            jax.ShapeDtypeStruct((N, 16), _f32),
        ],
    )(score, g, r_t, t2)


def _tc_pred_t(r_t, d1W, d1b, d2W, d2b, d3W, d3b):
    def body(rt_ref, w1, b1, w2, b2, w3, b3, o_ref):
        o_ref[...] = jax.nn.sigmoid(
            _mlp3_(rt_ref[...], w1[...], b1[...], w2[...], b2[...],
                   w3[...], b3[...]))

    wspec = pl.BlockSpec((H, H), lambda i: (0, 0))
    bspec = pl.BlockSpec((1, H), lambda i: (0, 0))
    return pl.pallas_call(
        body,
        grid=(_GRID,),
        in_specs=[pl.BlockSpec((BN, H), lambda i: (i, 0)),
                  wspec, bspec, wspec, bspec,
                  pl.BlockSpec((H, 1), lambda i: (0, 0)),
                  pl.BlockSpec((1, 1), lambda i: (0, 0))],
        out_specs=pl.BlockSpec((BN, 1), lambda i: (i, 0)),
        out_shape=jax.ShapeDtypeStruct((N, 1), _f32),
    )(r_t, d1W, d1b, d2W, d2b, d3W, d3b)


def _tc_final(h4, s2, r_o, t2, We, be, p1, p0):
    p1aW, p1ab, p1bW, p1bb, p1cW, p1cb = p1
    p0aW, p0ab, p0bW, p0bb, p0cW, p0cb = p0

    def body(h_ref, s_ref, ro_ref, t_ref, wer, weh, be_ref,
             w1a, b1a, w1b, b1b, w1c, b1c,
             w0a, b0a, w0b, b0b, w0c, b0c, z2_ref, pred_ref):
        ssum = s_ref[0, 0, :, 0] + s_ref[0, 1, :, 0]
        rec = (1.0 / (ssum + 1e-9))[:, None]
        h_lo = (h_ref[0, 0, 0] + h_ref[0, 1, 0]) * rec
        h_hi = (h_ref[0, 0, 1] + h_ref[0, 1, 1]) * rec
        hrow = jnp.concatenate([h_lo, h_hi], axis=1)
        z2 = (_dot(ro_ref[...], wer[...]) + _dot(hrow, weh[...])
              + be_ref[...])
        z2_ref[...] = z2
        pv1 = _mlp3_(z2, w1a[...], b1a[...], w1b[...], b1b[...],
                     w1c[...], b1c[...])
        pv0 = _mlp3_(z2, w0a[...], b0a[...], w0b[...], b0b[...],
                     w0c[...], b0c[...])
        pred_ref[...] = jnp.where(t_ref[...] > 0, pv1, pv0)

    wspec = pl.BlockSpec((H, H), lambda i: (0, 0))
    bspec = pl.BlockSpec((1, H), lambda i: (0, 0))
    cspec = pl.BlockSpec((H, 1), lambda i: (0, 0))
    sspec = pl.BlockSpec((1, 1), lambda i: (0, 0))
    return pl.pallas_call(
        body,
        grid=(_GRID,),
        in_specs=[
            pl.BlockSpec((1, NC, 2, BN, HH), lambda i: (0, 0, 0, i, 0)),
            pl.BlockSpec((1, NC, BN, 16), lambda i: (0, 0, i, 0)),
            pl.BlockSpec((BN, H), lambda i: (i, 0)),
            pl.BlockSpec((BN, 1), lambda i: (i, 0)),
            wspec, wspec, bspec,
            wspec, bspec, wspec, bspec, cspec, sspec,
            wspec, bspec, wspec, bspec, cspec, sspec,
        ],
        out_specs=[
            pl.BlockSpec((BN, H), lambda i: (i, 0)),
            pl.BlockSpec((BN, 1), lambda i: (i, 0)),
        ],
        out_shape=[
            jax.ShapeDtypeStruct((N, H), _f32),
            jax.ShapeDtypeStruct((N, 1), _f32),
        ],
    )(h4[None], s2[None], r_o, t2, We[:H], We[H:], be,
      p1aW, p1ab, p1bW, p1bb, p1cW, p1cb,
      p0aW, p0ab, p0bW, p0bb, p0cW, p0cb)


def kernel(x, t, z, edge_index, Wgo, bgo, Wgt, bgt, Wa, ba, We, be,
           d1W, d1b, d2W, d2b, d3W, d3b, p1aW, p1ab, p1bW, p1bb, p1cW,
           p1cb, p0aW, p0ab, p0bW, p0bb, p0cW, p0cb):
    src = edge_index[0]
    dst = edge_index[1]
    t2 = t[:, None]
    Wg2 = jnp.stack([Wgo, Wgt])
    bgo2, bgt2, be2 = bgo[None], bgt[None], be[None]
    ba2 = ba[None]
    d1b2, d2b2, d3b2 = d1b[None], d2b[None], d3b[None]
    p1ab2, p1bb2, p1cb2 = p1ab[None], p1bb[None], p1cb[None]
    p0ab2, p0bb2, p0cb2 = p0ab[None], p0bb[None], p0cb[None]

    pad = EP - E
    srcp = jnp.concatenate([src, jnp.zeros((pad,), jnp.int32)])
    dstp = jnp.concatenate([dst, jnp.full((pad,), N, jnp.int32)])
    src2 = srcp.reshape(EP // K, K)
    dst2 = dstp.reshape(EP // K, K)
    deg2 = _sc_degree(dst2)
    xwp4 = _tc_xwp(x, Wg2, deg2)
    agg4 = _sc_gcn_agg(xwp4, src2, dst2)
    r_o, r_t, score, g = _tc_gcn_post(agg4, xwp4, deg2, bgo2, bgt2,
                                      Wa, ba2)
    m2, ms_rep = _tc_msg(score, g, t2, r_t)
    h4 = _sc_attn_rows(m2, src2, dst2)
    s2 = _sc_attn_denom(ms_rep, src2, dst2)
    pred_t = _tc_pred_t(r_t, d1W, d1b2, d2W, d2b2, d3W, d3b2)
    z2, pred = _tc_final(h4, s2, r_o, t2, We, be2,
                         (p1aW, p1ab2, p1bW, p1bb2, p1cW, p1cb2),
                         (p0aW, p0ab2, p0bW, p0bb2, p0cW, p0cb2))
    return (pred_t, pred, z2)


# speedup vs baseline: 1.4228x; 1.4228x over previous
import functools

import jax
import jax.numpy as jnp
from jax import lax
from jax.experimental import pallas as pl
from jax.experimental.pallas import tpu as pltpu
from jax.experimental.pallas import tpu_sc as plsc

N = 10000
E = 320000
H = 128
HH = H // 2
NC = 2
NS = 16
K = 128
EP = 323584
NROW = 10240
RPT = NROW // NS

_f32 = jnp.float32


def _mesh():
    return plsc.VectorSubcoreMesh(core_axis_name="c", subcore_axis_name="s")


_SC_PARAMS = pltpu.CompilerParams(use_tc_tiling_on_sc=False)


def _zero_vmem2d(buf, rows, width):
    @pl.loop(0, rows)
    def _(i):
        @pl.loop(0, width, step=16)
        def _(j):
            buf[i, pl.ds(j, 16)] = jnp.zeros((16,), _f32)


def _fill_ones2d(buf, rows, width):
    @pl.loop(0, rows)
    def _(i):
        @pl.loop(0, width, step=16)
        def _(j):
            buf[i, pl.ds(j, 16)] = jnp.ones((16,), _f32)


def _zero_shared_rows(zbuf, zrows, table, row0, nrows):
    full, rem = nrows // zrows, nrows % zrows
    for q in range(full):
        pltpu.sync_copy(zbuf, table.at[pl.ds(row0 + q * zrows, zrows)])
    if rem:
        pltpu.sync_copy(zbuf.at[pl.ds(0, rem)],
                        table.at[pl.ds(row0 + full * zrows, rem)])


def _sc_degree(dst2):
    nchunk = EP // (K * NC * NS)

    @functools.partial(
        pl.kernel,
        out_type=jax.ShapeDtypeStruct((NC, NROW, 16), _f32),
        mesh=_mesh(),
        compiler_params=_SC_PARAMS,
        scratch_types=[
            pltpu.VMEM((nchunk, K), jnp.int32),
            pltpu.VMEM((K, 16), _f32),
            pltpu.VMEM((80, 16), _f32),
            pltpu.VMEM((160, 16), _f32),
            pltpu.VMEM_SHARED((NROW, 16), _f32),
            pltpu.SemaphoreType.DMA,
        ],
    )
    def k(dst_hbm, out_hbm, dst2_v, ones_v, zb_v, stage_v, table_sh, sem):
        c = lax.axis_index("c")
        s = lax.axis_index("s")
        wid = c * NS + s
        _fill_ones2d(ones_v, K, 16)
        _zero_vmem2d(zb_v, 80, 16)
        _zero_shared_rows(zb_v, 80, table_sh, s * RPT, RPT)
        pltpu.sync_copy(dst_hbm.at[pl.ds(wid * nchunk, nchunk)], dst2_v)
        plsc.subcore_barrier()

        @pl.loop(0, nchunk)
        def _(j):
            pltpu.async_copy(ones_v, table_sh.at[dst2_v.at[j]], sem,
                             add=True)

        @pl.loop(0, nchunk)
        def _(j):
            pltpu.make_async_copy(ones_v, table_sh.at[dst2_v.at[0]],
                                  sem).wait()

        plsc.subcore_barrier()
        for half in range(4):
            r0 = s * RPT + half * 160
            pltpu.sync_copy(table_sh.at[pl.ds(r0, 160)], stage_v)
            pltpu.sync_copy(stage_v, out_hbm.at[c, pl.ds(r0, 160)])

    return k(dst2)


def _add_offset(idx2_v, nrows, off):
    @pl.loop(0, nrows)
    def _(i):
        @pl.loop(0, K, step=16)
        def _(j):
            idx2_v[i, pl.ds(j, 16)] = idx2_v[i, pl.ds(j, 16)] + off


def _pipe_gather_scatter(nchunk, streams):
    def start_g(j, b):
        for (src_hbm, idx2_v, _, bufs, gsems, _ss, _t) in streams:
            pltpu.async_copy(src_hbm.at[idx2_v.at[j]], bufs[b], gsems[b])

    def wait_g(b):
        for (src_hbm, idx2_v, _, bufs, gsems, _ss, _t) in streams:
            pltpu.make_async_copy(src_hbm.at[idx2_v.at[0]], bufs[b],
                                  gsems[b]).wait()

    def scat(j, b):
        for (_s, _i, dst2_v, bufs, _gs, _ss, table_sh) in streams:
            pltpu.sync_copy(bufs[b], table_sh.at[dst2_v.at[j]], add=True)

    start_g(0, 0)
    start_g(1, 1)

    @pl.loop(0, nchunk, step=2)
    def _(j):
        wait_g(0)
        scat(j, 0)

        @pl.when(j + 2 < nchunk)
        def _():
            start_g(j + 2, 0)

        @pl.when(j + 1 < nchunk)
        def _():
            wait_g(1)
            scat(j + 1, 1)

        @pl.when(j + 3 < nchunk)
        def _():
            start_g(j + 3, 1)


def _sc_gcn_agg(xwp4, src2, dst2):
    nchunk = EP // (K * NS)

    @functools.partial(
        pl.kernel,
        out_type=jax.ShapeDtypeStruct((NC, 2, NROW, HH), _f32),
        mesh=_mesh(),
        compiler_params=_SC_PARAMS,
        scratch_types=[
            pltpu.VMEM((nchunk, K), jnp.int32),
            pltpu.VMEM((nchunk, K), jnp.int32),
            pltpu.VMEM((K, HH), _f32),
            pltpu.VMEM((K, HH), _f32),
            pltpu.VMEM((160, HH), _f32),
            pltpu.VMEM_SHARED((NROW, HH), _f32),
            pltpu.SemaphoreType.DMA,
            pltpu.SemaphoreType.DMA,
            pltpu.SemaphoreType.DMA,
            pltpu.SemaphoreType.DMA,
        ],
    )
    def k(x_hbm, src_hbm, dst_hbm, out_hbm, src2_v, dst2_v, buf0, buf1,
          stage_v, table_sh, sem0, sem1, sem2, sem3):
        c = lax.axis_index("c")
        s = lax.axis_index("s")
        pltpu.sync_copy(src_hbm.at[pl.ds(s * nchunk, nchunk)], src2_v)
        pltpu.sync_copy(dst_hbm.at[pl.ds(s * nchunk, nchunk)], dst2_v)
        _add_offset(src2_v, nchunk, c * (2 * N))
        for q in range(2):
            if q == 1:
                _add_offset(src2_v, nchunk, N)
            _zero_vmem2d(buf0, K, HH)
            _zero_shared_rows(buf0, K, table_sh, s * RPT, RPT)
            plsc.subcore_barrier()
            _pipe_gather_scatter(
                nchunk,
                [(x_hbm, src2_v, dst2_v, (buf0, buf1), (sem0, sem1),
                  (sem2, sem3), table_sh)])
            plsc.subcore_barrier()
            for half in range(4):
                r0 = s * RPT + half * 160
                pltpu.sync_copy(table_sh.at[pl.ds(r0, 160)], stage_v)
                pltpu.sync_copy(stage_v, out_hbm.at[c, q, pl.ds(r0, 160)])

    return k(xwp4, src2, dst2)


HA = HH + 16


def _sc_attn(m2, src2, dst2):
    nchunk = EP // (K * NS)

    @functools.partial(
        pl.kernel,
        out_type=jax.ShapeDtypeStruct((NC, NROW, HA), _f32),
        mesh=_mesh(),
        compiler_params=_SC_PARAMS,
        scratch_types=[
            pltpu.VMEM((nchunk, K), jnp.int32),
            pltpu.VMEM((nchunk, K), jnp.int32),
            pltpu.VMEM((K, HA), _f32),
            pltpu.VMEM((K, HA), _f32),
            pltpu.VMEM((160, HA), _f32),
            pltpu.VMEM_SHARED((NROW, HA), _f32),
            pltpu.SemaphoreType.DMA,
            pltpu.SemaphoreType.DMA,
            pltpu.SemaphoreType.DMA,
            pltpu.SemaphoreType.DMA,
        ],
    )
    def k(m_hbm, src_hbm, dst_hbm, out_hbm, src2_v, dst2_v, buf0, buf1,
          stage_v, table_sh, sem0, sem1, sem2, sem3):
        c = lax.axis_index("c")
        s = lax.axis_index("s")
        pltpu.sync_copy(src_hbm.at[pl.ds(s * nchunk, nchunk)], src2_v)
        pltpu.sync_copy(dst_hbm.at[pl.ds(s * nchunk, nchunk)], dst2_v)
        _add_offset(src2_v, nchunk, c * N)
        _zero_vmem2d(buf0, K, HA)
        _zero_shared_rows(buf0, K, table_sh, s * RPT, RPT)
        plsc.subcore_barrier()
        _pipe_gather_scatter(
            nchunk,
            [(m_hbm, src2_v, dst2_v, (buf0, buf1), (sem0, sem1),
              (sem2, sem3), table_sh)])
        plsc.subcore_barrier()
        for half in range(4):
            r0 = s * RPT + half * 160
            pltpu.sync_copy(table_sh.at[pl.ds(r0, 160)], stage_v)
            pltpu.sync_copy(stage_v, out_hbm.at[c, pl.ds(r0, 160)])

    return k(m2, src2, dst2)


BN = 1000
_GRID = N // BN
_HI = lax.Precision.HIGHEST


def _dot(a, b):
    return jnp.dot(a, b, preferred_element_type=_f32, precision=_HI)


def _lrelu_(v):
    return jnp.where(v > 0, v, 0.2 * v)


def _mlp3_(v, W1, b1, W2, b2, W3, b3):
    v = _lrelu_(_dot(v, W1) + b1)
    v = _lrelu_(_dot(v, W2) + b2)
    return _dot(v, W3) + b3


def _tc_xwp(x, Wg2, deg2):
    def body(x_ref, w_ref, deg_ref, o_ref):
        q = (pl.program_id(0) // _GRID) % 2
        deg = deg_ref[0, 0, :, 0] + deg_ref[0, 1, :, 0] + 1.0
        dinv = lax.rsqrt(deg)[:, None]
        xw = _dot(x_ref[...], w_ref[0])
        half = jnp.where(q == 0, xw[:, :HH], xw[:, HH:])
        o_ref[...] = half * dinv

    return pl.pallas_call(
        body,
        grid=(4 * _GRID,),
        in_specs=[
            pl.BlockSpec((BN, H), lambda i: (i % _GRID, 0)),
            pl.BlockSpec((1, H, H), lambda i: (i // (2 * _GRID), 0, 0)),
            pl.BlockSpec((1, NC, BN, 16), lambda i: (0, 0, i % _GRID, 0)),
        ],
        out_specs=pl.BlockSpec((BN, HH), lambda i: (i, 0)),
        out_shape=jax.ShapeDtypeStruct((4 * N, HH), _f32),
    )(x, Wg2, deg2[None])


def _tc_gcn_post(agg4, xwp4, deg2, bgo, bgt, Wa, ba):
    def body(agg_ref, xo0_ref, xo1_ref, xt0_ref, xt1_ref, deg_ref,
             bgo_ref, bgt_ref, wao_ref, wat_ref, ba_ref,
             ro_ref, rt_ref, sc_ref, g_ref):
        i = pl.program_id(0)
        deg = deg_ref[0, 0, :, 0] + deg_ref[0, 1, :, 0] + 1.0
        dinv = lax.rsqrt(deg)[:, None]
        agg_o = jnp.concatenate([agg_ref[0, 0, 0], agg_ref[0, 0, 1]], axis=1)
        agg_t = jnp.concatenate([agg_ref[0, 1, 0], agg_ref[0, 1, 1]], axis=1)
        xwp_o = jnp.concatenate([xo0_ref[...], xo1_ref[...]], axis=1)
        xwp_t = jnp.concatenate([xt0_ref[...], xt1_ref[...]], axis=1)
        ro = jnp.maximum((agg_o + xwp_o) * dinv + bgo_ref[...], 0.0)
        rt = jnp.maximum((agg_t + xwp_t) * dinv + bgt_ref[...], 0.0)
        sc = _lrelu_(_dot(ro, wao_ref[...]) + _dot(rt, wat_ref[...])
                     + ba_ref[...])
        ro_ref[...] = ro
        rt_ref[...] = rt
        sc_ref[...] = sc

        @pl.when(i == 0)
        def _():
            g_ref[...] = jnp.full((1, 1), -jnp.inf, _f32)

        g_ref[...] = jnp.maximum(g_ref[...], jnp.max(sc))

    hspec = pl.BlockSpec((BN, HH), lambda i: (i, 0))
    return pl.pallas_call(
        body,
        grid=(_GRID,),
        in_specs=[
            pl.BlockSpec((1, NC, 2, BN, HH), lambda i: (0, 0, 0, i, 0)),
            pl.BlockSpec((BN, HH), lambda i: (i, 0)),
            pl.BlockSpec((BN, HH), lambda i: (_GRID + i, 0)),
            pl.BlockSpec((BN, HH), lambda i: (2 * _GRID + i, 0)),
            pl.BlockSpec((BN, HH), lambda i: (3 * _GRID + i, 0)),
            pl.BlockSpec((1, NC, BN, 16), lambda i: (0, 0, i, 0)),
            pl.BlockSpec((1, H), lambda i: (0, 0)),
            pl.BlockSpec((1, H), lambda i: (0, 0)),
            pl.BlockSpec((H, 1), lambda i: (0, 0)),
            pl.BlockSpec((H, 1), lambda i: (0, 0)),
            pl.BlockSpec((1, 1), lambda i: (0, 0)),
        ],
        out_specs=[
            pl.BlockSpec((BN, H), lambda i: (i, 0)),
            pl.BlockSpec((BN, H), lambda i: (i, 0)),
            pl.BlockSpec((BN, 1), lambda i: (i, 0)),
            pl.BlockSpec((1, 1), lambda i: (0, 0)),
        ],
        out_shape=[
            jax.ShapeDtypeStruct((N, H), _f32),
            jax.ShapeDtypeStruct((N, H), _f32),
            jax.ShapeDtypeStruct((N, 1), _f32),
            jax.ShapeDtypeStruct((1, 1), _f32),
        ],
    )(agg4[None], xwp4, xwp4, xwp4, xwp4, deg2[None], bgo, bgt,
      Wa[:H], Wa[H:], ba)


def _tc_msg(score, g, t2, r_t):
    def body(sc_ref, g_ref, rt_ref, t_ref, m_ref):
        q = pl.program_id(0) // _GRID
        ms = jnp.where(t_ref[...] > 0,
                       jnp.exp(sc_ref[...] - g_ref[...]), 0.0)
        rt = rt_ref[...]
        half = jnp.where(q == 0, rt[:, :HH], rt[:, HH:])
        m_ref[...] = jnp.concatenate(
            [ms * half, jnp.broadcast_to(ms, (BN, 16))], axis=1)

    return pl.pallas_call(
        body,
        grid=(2 * _GRID,),
        in_specs=[
            pl.BlockSpec((BN, 1), lambda i: (i % _GRID, 0)),
            pl.BlockSpec((1, 1), lambda i: (0, 0)),
            pl.BlockSpec((BN, H), lambda i: (i % _GRID, 0)),
            pl.BlockSpec((BN, 1), lambda i: (i % _GRID, 0)),
        ],
        out_specs=pl.BlockSpec((BN, HA), lambda i: (i, 0)),
        out_shape=jax.ShapeDtypeStruct((2 * N, HA), _f32),
    )(score, g, r_t, t2)


def _tc_pred_t(r_t, d1W, d1b, d2W, d2b, d3W, d3b):
    def body(rt_ref, w1, b1, w2, b2, w3, b3, o_ref):
        o_ref[...] = jax.nn.sigmoid(
            _mlp3_(rt_ref[...], w1[...], b1[...], w2[...], b2[...],
                   w3[...], b3[...]))

    wspec = pl.BlockSpec((H, H), lambda i: (0, 0))
    bspec = pl.BlockSpec((1, H), lambda i: (0, 0))
    return pl.pallas_call(
        body,
        grid=(_GRID,),
        in_specs=[pl.BlockSpec((BN, H), lambda i: (i, 0)),
                  wspec, bspec, wspec, bspec,
                  pl.BlockSpec((H, 1), lambda i: (0, 0)),
                  pl.BlockSpec((1, 1), lambda i: (0, 0))],
        out_specs=pl.BlockSpec((BN, 1), lambda i: (i, 0)),
        out_shape=jax.ShapeDtypeStruct((N, 1), _f32),
    )(r_t, d1W, d1b, d2W, d2b, d3W, d3b)


def _tc_final(h4, r_o, t2, We, be, p1, p0):
    p1aW, p1ab, p1bW, p1bb, p1cW, p1cb = p1
    p0aW, p0ab, p0bW, p0bb, p0cW, p0cb = p0

    def body(h_ref, ro_ref, t_ref, wer, weh, be_ref,
             w1a, b1a, w1b, b1b, w1c, b1c,
             w0a, b0a, w0b, b0b, w0c, b0c, z2_ref, pred_ref):
        ssum = h_ref[0, 0, :, HH]
        rec = (1.0 / (ssum + 1e-9))[:, None]
        h_lo = h_ref[0, 0, :, :HH] * rec
        h_hi = h_ref[0, 1, :, :HH] * rec
        hrow = jnp.concatenate([h_lo, h_hi], axis=1)
        z2 = (_dot(ro_ref[...], wer[...]) + _dot(hrow, weh[...])
              + be_ref[...])
        z2_ref[...] = z2
        pv1 = _mlp3_(z2, w1a[...], b1a[...], w1b[...], b1b[...],
                     w1c[...], b1c[...])
        pv0 = _mlp3_(z2, w0a[...], b0a[...], w0b[...], b0b[...],
                     w0c[...], b0c[...])
        pred_ref[...] = jnp.where(t_ref[...] > 0, pv1, pv0)

    wspec = pl.BlockSpec((H, H), lambda i: (0, 0))
    bspec = pl.BlockSpec((1, H), lambda i: (0, 0))
    cspec = pl.BlockSpec((H, 1), lambda i: (0, 0))
    sspec = pl.BlockSpec((1, 1), lambda i: (0, 0))
    return pl.pallas_call(
        body,
        grid=(_GRID,),
        in_specs=[
            pl.BlockSpec((1, NC, BN, HA), lambda i: (0, 0, i, 0)),
            pl.BlockSpec((BN, H), lambda i: (i, 0)),
            pl.BlockSpec((BN, 1), lambda i: (i, 0)),
            wspec, wspec, bspec,
            wspec, bspec, wspec, bspec, cspec, sspec,
            wspec, bspec, wspec, bspec, cspec, sspec,
        ],
        out_specs=[
            pl.BlockSpec((BN, H), lambda i: (i, 0)),
            pl.BlockSpec((BN, 1), lambda i: (i, 0)),
        ],
        out_shape=[
            jax.ShapeDtypeStruct((N, H), _f32),
            jax.ShapeDtypeStruct((N, 1), _f32),
        ],
    )(h4[None], r_o, t2, We[:H], We[H:], be,
      p1aW, p1ab, p1bW, p1bb, p1cW, p1cb,
      p0aW, p0ab, p0bW, p0bb, p0cW, p0cb)


def kernel(x, t, z, edge_index, Wgo, bgo, Wgt, bgt, Wa, ba, We, be,
           d1W, d1b, d2W, d2b, d3W, d3b, p1aW, p1ab, p1bW, p1bb, p1cW,
           p1cb, p0aW, p0ab, p0bW, p0bb, p0cW, p0cb):
    src = edge_index[0]
    dst = edge_index[1]
    t2 = t[:, None]
    Wg2 = jnp.stack([Wgo, Wgt])
    bgo2, bgt2, be2 = bgo[None], bgt[None], be[None]
    ba2 = ba[None]
    d1b2, d2b2, d3b2 = d1b[None], d2b[None], d3b[None]
    p1ab2, p1bb2, p1cb2 = p1ab[None], p1bb[None], p1cb[None]
    p0ab2, p0bb2, p0cb2 = p0ab[None], p0bb[None], p0cb[None]

    pad = EP - E
    srcp = jnp.concatenate([src, jnp.zeros((pad,), jnp.int32)])
    dstp = jnp.concatenate([dst, jnp.full((pad,), N, jnp.int32)])
    src2 = srcp.reshape(EP // K, K)
    dst2 = dstp.reshape(EP // K, K)
    deg2 = _sc_degree(dst2)
    xwp4 = _tc_xwp(x, Wg2, deg2)
    agg4 = _sc_gcn_agg(xwp4, src2, dst2)
    r_o, r_t, score, g = _tc_gcn_post(agg4, xwp4, deg2, bgo2, bgt2,
                                      Wa, ba2)
    m2 = _tc_msg(score, g, t2, r_t)
    h4 = _sc_attn(m2, src2, dst2)
    pred_t = _tc_pred_t(r_t, d1W, d1b2, d2W, d2b2, d3W, d3b2)
    z2, pred = _tc_final(h4, r_o, t2, We, be2,
                         (p1aW, p1ab2, p1bW, p1bb2, p1cW, p1cb2),
                         (p0aW, p0ab2, p0bW, p0bb2, p0cW, p0cb2))
    return (pred_t, pred, z2)


# speedup vs baseline: 1.5735x; 1.1059x over previous
import functools

import jax
import jax.numpy as jnp
from jax import lax
from jax.experimental import pallas as pl
from jax.experimental.pallas import tpu as pltpu
from jax.experimental.pallas import tpu_sc as plsc

N = 10000
E = 320000
H = 128
HH = H // 2
NC = 2
NS = 16
K = 128
EP = 323584
NROW = 10240
RPT = NROW // NS

_f32 = jnp.float32


def _mesh():
    return plsc.VectorSubcoreMesh(core_axis_name="c", subcore_axis_name="s")


_SC_PARAMS = pltpu.CompilerParams(use_tc_tiling_on_sc=False)


def _zero_vmem2d(buf, rows, width):
    @pl.loop(0, rows)
    def _(i):
        @pl.loop(0, width, step=16)
        def _(j):
            buf[i, pl.ds(j, 16)] = jnp.zeros((16,), _f32)


def _fill_ones2d(buf, rows, width):
    @pl.loop(0, rows)
    def _(i):
        @pl.loop(0, width, step=16)
        def _(j):
            buf[i, pl.ds(j, 16)] = jnp.ones((16,), _f32)


def _zero_shared_rows(zbuf, zrows, table, row0, nrows):
    full, rem = nrows // zrows, nrows % zrows
    for q in range(full):
        pltpu.sync_copy(zbuf, table.at[pl.ds(row0 + q * zrows, zrows)])
    if rem:
        pltpu.sync_copy(zbuf.at[pl.ds(0, rem)],
                        table.at[pl.ds(row0 + full * zrows, rem)])


def _sc_degree(dst2):
    nchunk = EP // (K * NC * NS)

    @functools.partial(
        pl.kernel,
        out_type=jax.ShapeDtypeStruct((NC, NROW, 16), _f32),
        mesh=_mesh(),
        compiler_params=_SC_PARAMS,
        scratch_types=[
            pltpu.VMEM((nchunk, K), jnp.int32),
            pltpu.VMEM((K, 16), _f32),
            pltpu.VMEM((80, 16), _f32),
            pltpu.VMEM((160, 16), _f32),
            pltpu.VMEM_SHARED((NROW, 16), _f32),
            pltpu.SemaphoreType.DMA,
        ],
    )
    def k(dst_hbm, out_hbm, dst2_v, ones_v, zb_v, stage_v, table_sh, sem):
        c = lax.axis_index("c")
        s = lax.axis_index("s")
        wid = c * NS + s
        _fill_ones2d(ones_v, K, 16)
        _zero_vmem2d(zb_v, 80, 16)
        _zero_shared_rows(zb_v, 80, table_sh, s * RPT, RPT)
        pltpu.sync_copy(dst_hbm.at[pl.ds(wid * nchunk, nchunk)], dst2_v)
        plsc.subcore_barrier()

        @pl.loop(0, nchunk)
        def _(j):
            pltpu.async_copy(ones_v, table_sh.at[dst2_v.at[j]], sem,
                             add=True)

        @pl.loop(0, nchunk)
        def _(j):
            pltpu.make_async_copy(ones_v, table_sh.at[dst2_v.at[0]],
                                  sem).wait()

        plsc.subcore_barrier()
        for half in range(4):
            r0 = s * RPT + half * 160
            pltpu.sync_copy(table_sh.at[pl.ds(r0, 160)], stage_v)
            pltpu.sync_copy(stage_v, out_hbm.at[c, pl.ds(r0, 160)])

    return k(dst2)


def _add_offset(idx2_v, nrows, off):
    @pl.loop(0, nrows)
    def _(i):
        @pl.loop(0, K, step=16)
        def _(j):
            idx2_v[i, pl.ds(j, 16)] = idx2_v[i, pl.ds(j, 16)] + off


def _pipe_gather_scatter(nchunk, streams):
    def start_g(j, b):
        for (src_hbm, idx2_v, _, bufs, gsems, _ss, _t) in streams:
            pltpu.async_copy(src_hbm.at[idx2_v.at[j]], bufs[b], gsems[b])

    def wait_g(b):
        for (src_hbm, idx2_v, _, bufs, gsems, _ss, _t) in streams:
            pltpu.make_async_copy(src_hbm.at[idx2_v.at[0]], bufs[b],
                                  gsems[b]).wait()

    def scat(j, b):
        for (_s, _i, dst2_v, bufs, _gs, _ss, table_sh) in streams:
            pltpu.sync_copy(bufs[b], table_sh.at[dst2_v.at[j]], add=True)

    start_g(0, 0)
    start_g(1, 1)

    @pl.loop(0, nchunk, step=2)
    def _(j):
        wait_g(0)
        scat(j, 0)

        @pl.when(j + 2 < nchunk)
        def _():
            start_g(j + 2, 0)

        @pl.when(j + 1 < nchunk)
        def _():
            wait_g(1)
            scat(j + 1, 1)

        @pl.when(j + 3 < nchunk)
        def _():
            start_g(j + 3, 1)


def _sc_gcn_agg(xwp4, src2, dst2):
    nchunk = EP // (K * NS)

    @functools.partial(
        pl.kernel,
        out_type=jax.ShapeDtypeStruct((NC, 2, NROW, HH), _f32),
        mesh=_mesh(),
        compiler_params=_SC_PARAMS,
        scratch_types=[
            pltpu.VMEM((nchunk, K), jnp.int32),
            pltpu.VMEM((nchunk, K), jnp.int32),
            pltpu.VMEM((K, HH), _f32),
            pltpu.VMEM((K, HH), _f32),
            pltpu.VMEM((160, HH), _f32),
            pltpu.VMEM_SHARED((NROW, HH), _f32),
            pltpu.SemaphoreType.DMA,
            pltpu.SemaphoreType.DMA,
            pltpu.SemaphoreType.DMA,
            pltpu.SemaphoreType.DMA,
        ],
    )
    def k(x_hbm, src_hbm, dst_hbm, out_hbm, src2_v, dst2_v, buf0, buf1,
          stage_v, table_sh, sem0, sem1, sem2, sem3):
        c = lax.axis_index("c")
        s = lax.axis_index("s")
        pltpu.sync_copy(src_hbm.at[pl.ds(s * nchunk, nchunk)], src2_v)
        pltpu.sync_copy(dst_hbm.at[pl.ds(s * nchunk, nchunk)], dst2_v)
        _add_offset(src2_v, nchunk, c * (2 * N))
        for q in range(2):
            if q == 1:
                _add_offset(src2_v, nchunk, N)
            _zero_vmem2d(buf0, K, HH)
            _zero_shared_rows(buf0, K, table_sh, s * RPT, RPT)
            plsc.subcore_barrier()
            _pipe_gather_scatter(
                nchunk,
                [(x_hbm, src2_v, dst2_v, (buf0, buf1), (sem0, sem1),
                  (sem2, sem3), table_sh)])
            plsc.subcore_barrier()
            for half in range(4):
                r0 = s * RPT + half * 160
                pltpu.sync_copy(table_sh.at[pl.ds(r0, 160)], stage_v)
                pltpu.sync_copy(stage_v, out_hbm.at[c, q, pl.ds(r0, 160)])

    return k(xwp4, src2, dst2)


HA = HH + 16


def _sc_attn(m2, src2, dst2):
    nchunk = EP // (K * NS)

    @functools.partial(
        pl.kernel,
        out_type=jax.ShapeDtypeStruct((NC, NROW, HA), _f32),
        mesh=_mesh(),
        compiler_params=_SC_PARAMS,
        scratch_types=[
            pltpu.VMEM((nchunk, K), jnp.int32),
            pltpu.VMEM((nchunk, K), jnp.int32),
            pltpu.VMEM((K, HA), _f32),
            pltpu.VMEM((K, HA), _f32),
            pltpu.VMEM((160, HA), _f32),
            pltpu.VMEM_SHARED((NROW, HA), _f32),
            pltpu.SemaphoreType.DMA,
            pltpu.SemaphoreType.DMA,
            pltpu.SemaphoreType.DMA,
            pltpu.SemaphoreType.DMA,
        ],
    )
    def k(m_hbm, src_hbm, dst_hbm, out_hbm, src2_v, dst2_v, buf0, buf1,
          stage_v, table_sh, sem0, sem1, sem2, sem3):
        c = lax.axis_index("c")
        s = lax.axis_index("s")
        pltpu.sync_copy(src_hbm.at[pl.ds(s * nchunk, nchunk)], src2_v)
        pltpu.sync_copy(dst_hbm.at[pl.ds(s * nchunk, nchunk)], dst2_v)
        _add_offset(src2_v, nchunk, c * N)
        _zero_vmem2d(buf0, K, HA)
        _zero_shared_rows(buf0, K, table_sh, s * RPT, RPT)
        plsc.subcore_barrier()
        _pipe_gather_scatter(
            nchunk,
            [(m_hbm, src2_v, dst2_v, (buf0, buf1), (sem0, sem1),
              (sem2, sem3), table_sh)])
        plsc.subcore_barrier()
        for half in range(4):
            r0 = s * RPT + half * 160
            pltpu.sync_copy(table_sh.at[pl.ds(r0, 160)], stage_v)
            pltpu.sync_copy(stage_v, out_hbm.at[c, pl.ds(r0, 160)])

    return k(m2, src2, dst2)


BN = 1000
_GRID = N // BN
_HI = lax.Precision.DEFAULT


def _dot(a, b):
    return jnp.dot(a, b, preferred_element_type=_f32, precision=_HI)


def _lrelu_(v):
    return jnp.where(v > 0, v, 0.2 * v)


def _mlp3_(v, W1, b1, W2, b2, W3, b3):
    v = _lrelu_(_dot(v, W1) + b1)
    v = _lrelu_(_dot(v, W2) + b2)
    return _dot(v, W3) + b3


def _tc_xwp(x, Wg2, deg2):
    def body(x_ref, w_ref, deg_ref, o_ref):
        q = (pl.program_id(0) // _GRID) % 2
        deg = deg_ref[0, 0, :, 0] + deg_ref[0, 1, :, 0] + 1.0
        dinv = lax.rsqrt(deg)[:, None]
        xw = _dot(x_ref[...], w_ref[0])
        half = jnp.where(q == 0, xw[:, :HH], xw[:, HH:])
        o_ref[...] = half * dinv

    return pl.pallas_call(
        body,
        grid=(4 * _GRID,),
        in_specs=[
            pl.BlockSpec((BN, H), lambda i: (i % _GRID, 0)),
            pl.BlockSpec((1, H, H), lambda i: (i // (2 * _GRID), 0, 0)),
            pl.BlockSpec((1, NC, BN, 16), lambda i: (0, 0, i % _GRID, 0)),
        ],
        out_specs=pl.BlockSpec((BN, HH), lambda i: (i, 0)),
        out_shape=jax.ShapeDtypeStruct((4 * N, HH), _f32),
    )(x, Wg2, deg2[None])


def _tc_gcn_post(agg4, xwp4, deg2, bgo, bgt, Wa, ba):
    def body(agg_ref, xo0_ref, xo1_ref, xt0_ref, xt1_ref, deg_ref,
             bgo_ref, bgt_ref, wao_ref, wat_ref, ba_ref,
             ro_ref, rt_ref, sc_ref, g_ref):
        i = pl.program_id(0)
        deg = deg_ref[0, 0, :, 0] + deg_ref[0, 1, :, 0] + 1.0
        dinv = lax.rsqrt(deg)[:, None]
        agg_o = jnp.concatenate([agg_ref[0, 0, 0], agg_ref[0, 0, 1]], axis=1)
        agg_t = jnp.concatenate([agg_ref[0, 1, 0], agg_ref[0, 1, 1]], axis=1)
        xwp_o = jnp.concatenate([xo0_ref[...], xo1_ref[...]], axis=1)
        xwp_t = jnp.concatenate([xt0_ref[...], xt1_ref[...]], axis=1)
        ro = jnp.maximum((agg_o + xwp_o) * dinv + bgo_ref[...], 0.0)
        rt = jnp.maximum((agg_t + xwp_t) * dinv + bgt_ref[...], 0.0)
        sc = _lrelu_(_dot(ro, wao_ref[...]) + _dot(rt, wat_ref[...])
                     + ba_ref[...])
        ro_ref[...] = ro
        rt_ref[...] = rt
        sc_ref[...] = sc

        @pl.when(i == 0)
        def _():
            g_ref[...] = jnp.full((1, 1), -jnp.inf, _f32)

        g_ref[...] = jnp.maximum(g_ref[...], jnp.max(sc))

    hspec = pl.BlockSpec((BN, HH), lambda i: (i, 0))
    return pl.pallas_call(
        body,
        grid=(_GRID,),
        in_specs=[
            pl.BlockSpec((1, NC, 2, BN, HH), lambda i: (0, 0, 0, i, 0)),
            pl.BlockSpec((BN, HH), lambda i: (i, 0)),
            pl.BlockSpec((BN, HH), lambda i: (_GRID + i, 0)),
            pl.BlockSpec((BN, HH), lambda i: (2 * _GRID + i, 0)),
            pl.BlockSpec((BN, HH), lambda i: (3 * _GRID + i, 0)),
            pl.BlockSpec((1, NC, BN, 16), lambda i: (0, 0, i, 0)),
            pl.BlockSpec((1, H), lambda i: (0, 0)),
            pl.BlockSpec((1, H), lambda i: (0, 0)),
            pl.BlockSpec((H, 1), lambda i: (0, 0)),
            pl.BlockSpec((H, 1), lambda i: (0, 0)),
            pl.BlockSpec((1, 1), lambda i: (0, 0)),
        ],
        out_specs=[
            pl.BlockSpec((BN, H), lambda i: (i, 0)),
            pl.BlockSpec((BN, H), lambda i: (i, 0)),
            pl.BlockSpec((BN, 1), lambda i: (i, 0)),
            pl.BlockSpec((1, 1), lambda i: (0, 0)),
        ],
        out_shape=[
            jax.ShapeDtypeStruct((N, H), _f32),
            jax.ShapeDtypeStruct((N, H), _f32),
            jax.ShapeDtypeStruct((N, 1), _f32),
            jax.ShapeDtypeStruct((1, 1), _f32),
        ],
    )(agg4[None], xwp4, xwp4, xwp4, xwp4, deg2[None], bgo, bgt,
      Wa[:H], Wa[H:], ba)


def _tc_msg(score, g, t2, r_t):
    def body(sc_ref, g_ref, rt_ref, t_ref, m_ref):
        q = pl.program_id(0) // _GRID
        ms = jnp.where(t_ref[...] > 0,
                       jnp.exp(sc_ref[...] - g_ref[...]), 0.0)
        rt = rt_ref[...]
        half = jnp.where(q == 0, rt[:, :HH], rt[:, HH:])
        m_ref[...] = jnp.concatenate(
            [ms * half, jnp.broadcast_to(ms, (BN, 16))], axis=1)

    return pl.pallas_call(
        body,
        grid=(2 * _GRID,),
        in_specs=[
            pl.BlockSpec((BN, 1), lambda i: (i % _GRID, 0)),
            pl.BlockSpec((1, 1), lambda i: (0, 0)),
            pl.BlockSpec((BN, H), lambda i: (i % _GRID, 0)),
            pl.BlockSpec((BN, 1), lambda i: (i % _GRID, 0)),
        ],
        out_specs=pl.BlockSpec((BN, HA), lambda i: (i, 0)),
        out_shape=jax.ShapeDtypeStruct((2 * N, HA), _f32),
    )(score, g, r_t, t2)


def _tc_pred_t(r_t, d1W, d1b, d2W, d2b, d3W, d3b):
    def body(rt_ref, w1, b1, w2, b2, w3, b3, o_ref):
        o_ref[...] = jax.nn.sigmoid(
            _mlp3_(rt_ref[...], w1[...], b1[...], w2[...], b2[...],
                   w3[...], b3[...]))

    wspec = pl.BlockSpec((H, H), lambda i: (0, 0))
    bspec = pl.BlockSpec((1, H), lambda i: (0, 0))
    return pl.pallas_call(
        body,
        grid=(_GRID,),
        in_specs=[pl.BlockSpec((BN, H), lambda i: (i, 0)),
                  wspec, bspec, wspec, bspec,
                  pl.BlockSpec((H, 1), lambda i: (0, 0)),
                  pl.BlockSpec((1, 1), lambda i: (0, 0))],
        out_specs=pl.BlockSpec((BN, 1), lambda i: (i, 0)),
        out_shape=jax.ShapeDtypeStruct((N, 1), _f32),
    )(r_t, d1W, d1b, d2W, d2b, d3W, d3b)


def _tc_final(h4, r_o, t2, We, be, p1, p0):
    p1aW, p1ab, p1bW, p1bb, p1cW, p1cb = p1
    p0aW, p0ab, p0bW, p0bb, p0cW, p0cb = p0

    def body(h_ref, ro_ref, t_ref, wer, weh, be_ref,
             w1a, b1a, w1b, b1b, w1c, b1c,
             w0a, b0a, w0b, b0b, w0c, b0c, z2_ref, pred_ref):
        ssum = h_ref[0, 0, :, HH]
        rec = (1.0 / (ssum + 1e-9))[:, None]
        h_lo = h_ref[0, 0, :, :HH] * rec
        h_hi = h_ref[0, 1, :, :HH] * rec
        hrow = jnp.concatenate([h_lo, h_hi], axis=1)
        z2 = (_dot(ro_ref[...], wer[...]) + _dot(hrow, weh[...])
              + be_ref[...])
        z2_ref[...] = z2
        pv1 = _mlp3_(z2, w1a[...], b1a[...], w1b[...], b1b[...],
                     w1c[...], b1c[...])
        pv0 = _mlp3_(z2, w0a[...], b0a[...], w0b[...], b0b[...],
                     w0c[...], b0c[...])
        pred_ref[...] = jnp.where(t_ref[...] > 0, pv1, pv0)

    wspec = pl.BlockSpec((H, H), lambda i: (0, 0))
    bspec = pl.BlockSpec((1, H), lambda i: (0, 0))
    cspec = pl.BlockSpec((H, 1), lambda i: (0, 0))
    sspec = pl.BlockSpec((1, 1), lambda i: (0, 0))
    return pl.pallas_call(
        body,
        grid=(_GRID,),
        in_specs=[
            pl.BlockSpec((1, NC, BN, HA), lambda i: (0, 0, i, 0)),
            pl.BlockSpec((BN, H), lambda i: (i, 0)),
            pl.BlockSpec((BN, 1), lambda i: (i, 0)),
            wspec, wspec, bspec,
            wspec, bspec, wspec, bspec, cspec, sspec,
            wspec, bspec, wspec, bspec, cspec, sspec,
        ],
        out_specs=[
            pl.BlockSpec((BN, H), lambda i: (i, 0)),
            pl.BlockSpec((BN, 1), lambda i: (i, 0)),
        ],
        out_shape=[
            jax.ShapeDtypeStruct((N, H), _f32),
            jax.ShapeDtypeStruct((N, 1), _f32),
        ],
    )(h4[None], r_o, t2, We[:H], We[H:], be,
      p1aW, p1ab, p1bW, p1bb, p1cW, p1cb,
      p0aW, p0ab, p0bW, p0bb, p0cW, p0cb)


def kernel(x, t, z, edge_index, Wgo, bgo, Wgt, bgt, Wa, ba, We, be,
           d1W, d1b, d2W, d2b, d3W, d3b, p1aW, p1ab, p1bW, p1bb, p1cW,
           p1cb, p0aW, p0ab, p0bW, p0bb, p0cW, p0cb):
    src = edge_index[0]
    dst = edge_index[1]
    t2 = t[:, None]
    Wg2 = jnp.stack([Wgo, Wgt])
    bgo2, bgt2, be2 = bgo[None], bgt[None], be[None]
    ba2 = ba[None]
    d1b2, d2b2, d3b2 = d1b[None], d2b[None], d3b[None]
    p1ab2, p1bb2, p1cb2 = p1ab[None], p1bb[None], p1cb[None]
    p0ab2, p0bb2, p0cb2 = p0ab[None], p0bb[None], p0cb[None]

    pad = EP - E
    srcp = jnp.concatenate([src, jnp.zeros((pad,), jnp.int32)])
    dstp = jnp.concatenate([dst, jnp.full((pad,), N, jnp.int32)])
    src2 = srcp.reshape(EP // K, K)
    dst2 = dstp.reshape(EP // K, K)
    deg2 = _sc_degree(dst2)
    xwp4 = _tc_xwp(x, Wg2, deg2)
    agg4 = _sc_gcn_agg(xwp4, src2, dst2)
    r_o, r_t, score, g = _tc_gcn_post(agg4, xwp4, deg2, bgo2, bgt2,
                                      Wa, ba2)
    m2 = _tc_msg(score, g, t2, r_t)
    h4 = _sc_attn(m2, src2, dst2)
    pred_t = _tc_pred_t(r_t, d1W, d1b2, d2W, d2b2, d3W, d3b2)
    z2, pred = _tc_final(h4, r_o, t2, We, be2,
                         (p1aW, p1ab2, p1bW, p1bb2, p1cW, p1cb2),
                         (p0aW, p0ab2, p0bW, p0bb2, p0cW, p0cb2))
    return (pred_t, pred, z2)


# speedup vs baseline: 1.6217x; 1.0306x over previous
import functools

import jax
import jax.numpy as jnp
from jax import lax
from jax.experimental import pallas as pl
from jax.experimental.pallas import tpu as pltpu
from jax.experimental.pallas import tpu_sc as plsc

N = 10000
E = 320000
H = 128
HH = H // 2
NC = 2
NS = 16
K = 128
EP = 323584
NROW = 10240
RPT = NROW // NS

_f32 = jnp.float32


def _mesh():
    return plsc.VectorSubcoreMesh(core_axis_name="c", subcore_axis_name="s")


_SC_PARAMS = pltpu.CompilerParams(use_tc_tiling_on_sc=False)


def _zero_vmem2d(buf, rows, width):
    @pl.loop(0, rows)
    def _(i):
        @pl.loop(0, width, step=16)
        def _(j):
            buf[i, pl.ds(j, 16)] = jnp.zeros((16,), _f32)


def _fill_ones2d(buf, rows, width):
    @pl.loop(0, rows)
    def _(i):
        @pl.loop(0, width, step=16)
        def _(j):
            buf[i, pl.ds(j, 16)] = jnp.ones((16,), _f32)


def _zero_shared_rows(zbuf, zrows, table, row0, nrows):
    full, rem = nrows // zrows, nrows % zrows
    for q in range(full):
        pltpu.sync_copy(zbuf, table.at[pl.ds(row0 + q * zrows, zrows)])
    if rem:
        pltpu.sync_copy(zbuf.at[pl.ds(0, rem)],
                        table.at[pl.ds(row0 + full * zrows, rem)])


def _sc_degree(dst2):
    nchunk = EP // (K * NC * NS)

    @functools.partial(
        pl.kernel,
        out_type=jax.ShapeDtypeStruct((NC, NROW, 16), _f32),
        mesh=_mesh(),
        compiler_params=_SC_PARAMS,
        scratch_types=[
            pltpu.VMEM((nchunk, K), jnp.int32),
            pltpu.VMEM((K, 16), _f32),
            pltpu.VMEM((80, 16), _f32),
            pltpu.VMEM((160, 16), _f32),
            pltpu.VMEM_SHARED((NROW, 16), _f32),
            pltpu.SemaphoreType.DMA,
        ],
    )
    def k(dst_hbm, out_hbm, dst2_v, ones_v, zb_v, stage_v, table_sh, sem):
        c = lax.axis_index("c")
        s = lax.axis_index("s")
        wid = c * NS + s
        _fill_ones2d(ones_v, K, 16)
        _zero_vmem2d(zb_v, 80, 16)
        _zero_shared_rows(zb_v, 80, table_sh, s * RPT, RPT)
        pltpu.sync_copy(dst_hbm.at[pl.ds(wid * nchunk, nchunk)], dst2_v)
        plsc.subcore_barrier()

        @pl.loop(0, nchunk)
        def _(j):
            pltpu.async_copy(ones_v, table_sh.at[dst2_v.at[j]], sem,
                             add=True)

        @pl.loop(0, nchunk)
        def _(j):
            pltpu.make_async_copy(ones_v, table_sh.at[dst2_v.at[0]],
                                  sem).wait()

        plsc.subcore_barrier()
        for half in range(4):
            r0 = s * RPT + half * 160
            pltpu.sync_copy(table_sh.at[pl.ds(r0, 160)], stage_v)
            pltpu.sync_copy(stage_v, out_hbm.at[c, pl.ds(r0, 160)])

    return k(dst2)


def _add_offset(idx2_v, nrows, off):
    @pl.loop(0, nrows)
    def _(i):
        @pl.loop(0, K, step=16)
        def _(j):
            idx2_v[i, pl.ds(j, 16)] = idx2_v[i, pl.ds(j, 16)] + off


def _pipe_gather_scatter(nchunk, streams):
    def start_g(j, b):
        for (src_hbm, idx2_v, _, bufs, gsems, _ss, _t) in streams:
            pltpu.async_copy(src_hbm.at[idx2_v.at[j]], bufs[b], gsems[b])

    def wait_g(b):
        for (src_hbm, idx2_v, _, bufs, gsems, _ss, _t) in streams:
            pltpu.make_async_copy(src_hbm.at[idx2_v.at[0]], bufs[b],
                                  gsems[b]).wait()

    def scat(j, b):
        for (_s, _i, dst2_v, bufs, _gs, _ss, table_sh) in streams:
            pltpu.sync_copy(bufs[b], table_sh.at[dst2_v.at[j]], add=True)

    start_g(0, 0)
    start_g(1, 1)

    @pl.loop(0, nchunk, step=2)
    def _(j):
        wait_g(0)
        scat(j, 0)

        @pl.when(j + 2 < nchunk)
        def _():
            start_g(j + 2, 0)

        @pl.when(j + 1 < nchunk)
        def _():
            wait_g(1)
            scat(j + 1, 1)

        @pl.when(j + 3 < nchunk)
        def _():
            start_g(j + 3, 1)


def _sc_gcn_agg(xwp4, src2, dst2):
    nchunk = EP // (K * NS)

    @functools.partial(
        pl.kernel,
        out_type=jax.ShapeDtypeStruct((NC, 2, NROW, HH), _f32),
        mesh=_mesh(),
        compiler_params=_SC_PARAMS,
        scratch_types=[
            pltpu.VMEM((nchunk, K), jnp.int32),
            pltpu.VMEM((nchunk, K), jnp.int32),
            pltpu.VMEM((K, HH), _f32),
            pltpu.VMEM((K, HH), _f32),
            pltpu.VMEM((160, HH), _f32),
            pltpu.VMEM_SHARED((NROW, HH), _f32),
            pltpu.SemaphoreType.DMA,
            pltpu.SemaphoreType.DMA,
            pltpu.SemaphoreType.DMA,
            pltpu.SemaphoreType.DMA,
        ],
    )
    def k(x_hbm, src_hbm, dst_hbm, out_hbm, src2_v, dst2_v, buf0, buf1,
          stage_v, table_sh, sem0, sem1, sem2, sem3):
        c = lax.axis_index("c")
        s = lax.axis_index("s")
        pltpu.sync_copy(src_hbm.at[pl.ds(s * nchunk, nchunk)], src2_v)
        pltpu.sync_copy(dst_hbm.at[pl.ds(s * nchunk, nchunk)], dst2_v)
        _add_offset(src2_v, nchunk, c * (2 * N))
        for q in range(2):
            if q == 1:
                _add_offset(src2_v, nchunk, N)
            _zero_vmem2d(buf0, K, HH)
            _zero_shared_rows(buf0, K, table_sh, s * RPT, RPT)
            plsc.subcore_barrier()
            _pipe_gather_scatter(
                nchunk,
                [(x_hbm, src2_v, dst2_v, (buf0, buf1), (sem0, sem1),
                  (sem2, sem3), table_sh)])
            plsc.subcore_barrier()
            for half in range(4):
                r0 = s * RPT + half * 160
                pltpu.sync_copy(table_sh.at[pl.ds(r0, 160)], stage_v)
                pltpu.sync_copy(stage_v, out_hbm.at[c, q, pl.ds(r0, 160)])

    return k(xwp4, src2, dst2)


HA = HH + 16


def _sc_attn(m2, src2, dst2):
    nchunk = EP // (K * NS)

    @functools.partial(
        pl.kernel,
        out_type=jax.ShapeDtypeStruct((NC, NROW, HA), _f32),
        mesh=_mesh(),
        compiler_params=_SC_PARAMS,
        scratch_types=[
            pltpu.VMEM((nchunk, K), jnp.int32),
            pltpu.VMEM((nchunk, K), jnp.int32),
            pltpu.VMEM((K, HA), _f32),
            pltpu.VMEM((K, HA), _f32),
            pltpu.VMEM((160, HA), _f32),
            pltpu.VMEM_SHARED((NROW, HA), _f32),
            pltpu.SemaphoreType.DMA,
            pltpu.SemaphoreType.DMA,
            pltpu.SemaphoreType.DMA,
            pltpu.SemaphoreType.DMA,
        ],
    )
    def k(m_hbm, src_hbm, dst_hbm, out_hbm, src2_v, dst2_v, buf0, buf1,
          stage_v, table_sh, sem0, sem1, sem2, sem3):
        c = lax.axis_index("c")
        s = lax.axis_index("s")
        pltpu.sync_copy(src_hbm.at[pl.ds(s * nchunk, nchunk)], src2_v)
        pltpu.sync_copy(dst_hbm.at[pl.ds(s * nchunk, nchunk)], dst2_v)
        _add_offset(src2_v, nchunk, c * N)
        _zero_vmem2d(buf0, K, HA)
        _zero_shared_rows(buf0, K, table_sh, s * RPT, RPT)
        plsc.subcore_barrier()
        _pipe_gather_scatter(
            nchunk,
            [(m_hbm, src2_v, dst2_v, (buf0, buf1), (sem0, sem1),
              (sem2, sem3), table_sh)])
        plsc.subcore_barrier()
        for half in range(4):
            r0 = s * RPT + half * 160
            pltpu.sync_copy(table_sh.at[pl.ds(r0, 160)], stage_v)
            pltpu.sync_copy(stage_v, out_hbm.at[c, pl.ds(r0, 160)])

    return k(m2, src2, dst2)


BN = 1000
_GRID = N // BN
_HI = lax.Precision.DEFAULT


def _dot(a, b):
    return jnp.dot(a, b, preferred_element_type=_f32, precision=_HI)


def _lrelu_(v):
    return jnp.where(v > 0, v, 0.2 * v)


def _mlp3_(v, W1, b1, W2, b2, W3, b3):
    v = _lrelu_(_dot(v, W1) + b1)
    v = _lrelu_(_dot(v, W2) + b2)
    return _dot(v, W3) + b3


def _tc_xwp(x, Wg2, deg2):
    def body(x_ref, w_ref, deg_ref, o_ref):
        q = (pl.program_id(0) // _GRID) % 2
        deg = deg_ref[0, 0, :, 0] + deg_ref[0, 1, :, 0] + 1.0
        dinv = lax.rsqrt(deg)[:, None]
        xw = _dot(x_ref[...], w_ref[0])
        half = jnp.where(q == 0, xw[:, :HH], xw[:, HH:])
        o_ref[...] = half * dinv

    return pl.pallas_call(
        body,
        grid=(4 * _GRID,),
        in_specs=[
            pl.BlockSpec((BN, H), lambda i: (i % _GRID, 0)),
            pl.BlockSpec((1, H, H), lambda i: (i // (2 * _GRID), 0, 0)),
            pl.BlockSpec((1, NC, BN, 16), lambda i: (0, 0, i % _GRID, 0)),
        ],
        out_specs=pl.BlockSpec((BN, HH), lambda i: (i, 0)),
        out_shape=jax.ShapeDtypeStruct((4 * N, HH), _f32),
    )(x, Wg2, deg2[None])


def _tc_gcn_post(agg4, xwp4, deg2, bgo, bgt, Wa, ba):
    def body(agg_ref, xo0_ref, xo1_ref, xt0_ref, xt1_ref, deg_ref,
             bgo_ref, bgt_ref, wao_ref, wat_ref, ba_ref,
             ro_ref, rt_ref, sc_ref, g_ref):
        i = pl.program_id(0)
        deg = deg_ref[0, 0, :, 0] + deg_ref[0, 1, :, 0] + 1.0
        dinv = lax.rsqrt(deg)[:, None]
        agg_o = jnp.concatenate([agg_ref[0, 0, 0], agg_ref[0, 0, 1]], axis=1)
        agg_t = jnp.concatenate([agg_ref[0, 1, 0], agg_ref[0, 1, 1]], axis=1)
        xwp_o = jnp.concatenate([xo0_ref[...], xo1_ref[...]], axis=1)
        xwp_t = jnp.concatenate([xt0_ref[...], xt1_ref[...]], axis=1)
        ro = jnp.maximum((agg_o + xwp_o) * dinv + bgo_ref[...], 0.0)
        rt = jnp.maximum((agg_t + xwp_t) * dinv + bgt_ref[...], 0.0)
        sc = _lrelu_(_dot(ro, wao_ref[...]) + _dot(rt, wat_ref[...])
                     + ba_ref[...])
        ro_ref[...] = ro
        rt_ref[...] = rt
        sc_ref[...] = sc

        @pl.when(i == 0)
        def _():
            g_ref[...] = jnp.full((1, 1), -jnp.inf, _f32)

        g_ref[...] = jnp.maximum(g_ref[...], jnp.max(sc))

    hspec = pl.BlockSpec((BN, HH), lambda i: (i, 0))
    return pl.pallas_call(
        body,
        grid=(_GRID,),
        in_specs=[
            pl.BlockSpec((1, NC, 2, BN, HH), lambda i: (0, 0, 0, i, 0)),
            pl.BlockSpec((BN, HH), lambda i: (i, 0)),
            pl.BlockSpec((BN, HH), lambda i: (_GRID + i, 0)),
            pl.BlockSpec((BN, HH), lambda i: (2 * _GRID + i, 0)),
            pl.BlockSpec((BN, HH), lambda i: (3 * _GRID + i, 0)),
            pl.BlockSpec((1, NC, BN, 16), lambda i: (0, 0, i, 0)),
            pl.BlockSpec((1, H), lambda i: (0, 0)),
            pl.BlockSpec((1, H), lambda i: (0, 0)),
            pl.BlockSpec((H, 1), lambda i: (0, 0)),
            pl.BlockSpec((H, 1), lambda i: (0, 0)),
            pl.BlockSpec((1, 1), lambda i: (0, 0)),
        ],
        out_specs=[
            pl.BlockSpec((BN, H), lambda i: (i, 0)),
            pl.BlockSpec((BN, H), lambda i: (i, 0)),
            pl.BlockSpec((BN, 1), lambda i: (i, 0)),
            pl.BlockSpec((1, 1), lambda i: (0, 0)),
        ],
        out_shape=[
            jax.ShapeDtypeStruct((N, H), _f32),
            jax.ShapeDtypeStruct((N, H), _f32),
            jax.ShapeDtypeStruct((N, 1), _f32),
            jax.ShapeDtypeStruct((1, 1), _f32),
        ],
    )(agg4[None], xwp4, xwp4, xwp4, xwp4, deg2[None], bgo, bgt,
      Wa[:H], Wa[H:], ba)


def _tc_msg(score, g, t2, r_t, d1W, d1b, d2W, d2b, d3W, d3b):
    def body(sc_ref, g_ref, rt_ref, t_ref, w1, b1, w2, b2, w3, b3,
             m_ref, pt_ref):
        q = pl.program_id(0) // _GRID
        ms = jnp.where(t_ref[...] > 0,
                       jnp.exp(sc_ref[...] - g_ref[...]), 0.0)
        rt = rt_ref[...]
        half = jnp.where(q == 0, rt[:, :HH], rt[:, HH:])
        m_ref[...] = jnp.concatenate(
            [ms * half, jnp.broadcast_to(ms, (BN, 16))], axis=1)
        pt_ref[...] = jax.nn.sigmoid(
            _mlp3_(rt, w1[...], b1[...], w2[...], b2[...], w3[...],
                   b3[...]))

    wspec = pl.BlockSpec((H, H), lambda i: (0, 0))
    bspec = pl.BlockSpec((1, H), lambda i: (0, 0))
    return pl.pallas_call(
        body,
        grid=(2 * _GRID,),
        in_specs=[
            pl.BlockSpec((BN, 1), lambda i: (i % _GRID, 0)),
            pl.BlockSpec((1, 1), lambda i: (0, 0)),
            pl.BlockSpec((BN, H), lambda i: (i % _GRID, 0)),
            pl.BlockSpec((BN, 1), lambda i: (i % _GRID, 0)),
            wspec, bspec, wspec, bspec,
            pl.BlockSpec((H, 1), lambda i: (0, 0)),
            pl.BlockSpec((1, 1), lambda i: (0, 0)),
        ],
        out_specs=[
            pl.BlockSpec((BN, HA), lambda i: (i, 0)),
            pl.BlockSpec((BN, 1), lambda i: (i % _GRID, 0)),
        ],
        out_shape=[
            jax.ShapeDtypeStruct((2 * N, HA), _f32),
            jax.ShapeDtypeStruct((N, 1), _f32),
        ],
    )(score, g, r_t, t2, d1W, d1b, d2W, d2b, d3W, d3b)


def _tc_final(h4, r_o, t2, We, be, p1, p0):
    p1aW, p1ab, p1bW, p1bb, p1cW, p1cb = p1
    p0aW, p0ab, p0bW, p0bb, p0cW, p0cb = p0

    def body(h_ref, ro_ref, t_ref, wer, weh, be_ref,
             w1a, b1a, w1b, b1b, w1c, b1c,
             w0a, b0a, w0b, b0b, w0c, b0c, z2_ref, pred_ref):
        ssum = h_ref[0, 0, :, HH]
        rec = (1.0 / (ssum + 1e-9))[:, None]
        h_lo = h_ref[0, 0, :, :HH] * rec
        h_hi = h_ref[0, 1, :, :HH] * rec
        hrow = jnp.concatenate([h_lo, h_hi], axis=1)
        z2 = (_dot(ro_ref[...], wer[...]) + _dot(hrow, weh[...])
              + be_ref[...])
        z2_ref[...] = z2
        pv1 = _mlp3_(z2, w1a[...], b1a[...], w1b[...], b1b[...],
                     w1c[...], b1c[...])
        pv0 = _mlp3_(z2, w0a[...], b0a[...], w0b[...], b0b[...],
                     w0c[...], b0c[...])
        pred_ref[...] = jnp.where(t_ref[...] > 0, pv1, pv0)

    wspec = pl.BlockSpec((H, H), lambda i: (0, 0))
    bspec = pl.BlockSpec((1, H), lambda i: (0, 0))
    cspec = pl.BlockSpec((H, 1), lambda i: (0, 0))
    sspec = pl.BlockSpec((1, 1), lambda i: (0, 0))
    return pl.pallas_call(
        body,
        grid=(_GRID,),
        in_specs=[
            pl.BlockSpec((1, NC, BN, HA), lambda i: (0, 0, i, 0)),
            pl.BlockSpec((BN, H), lambda i: (i, 0)),
            pl.BlockSpec((BN, 1), lambda i: (i, 0)),
            wspec, wspec, bspec,
            wspec, bspec, wspec, bspec, cspec, sspec,
            wspec, bspec, wspec, bspec, cspec, sspec,
        ],
        out_specs=[
            pl.BlockSpec((BN, H), lambda i: (i, 0)),
            pl.BlockSpec((BN, 1), lambda i: (i, 0)),
        ],
        out_shape=[
            jax.ShapeDtypeStruct((N, H), _f32),
            jax.ShapeDtypeStruct((N, 1), _f32),
        ],
    )(h4[None], r_o, t2, We[:H], We[H:], be,
      p1aW, p1ab, p1bW, p1bb, p1cW, p1cb,
      p0aW, p0ab, p0bW, p0bb, p0cW, p0cb)


def kernel(x, t, z, edge_index, Wgo, bgo, Wgt, bgt, Wa, ba, We, be,
           d1W, d1b, d2W, d2b, d3W, d3b, p1aW, p1ab, p1bW, p1bb, p1cW,
           p1cb, p0aW, p0ab, p0bW, p0bb, p0cW, p0cb):
    src = edge_index[0]
    dst = edge_index[1]
    t2 = t[:, None]
    Wg2 = jnp.stack([Wgo, Wgt])
    bgo2, bgt2, be2 = bgo[None], bgt[None], be[None]
    ba2 = ba[None]
    d1b2, d2b2, d3b2 = d1b[None], d2b[None], d3b[None]
    p1ab2, p1bb2, p1cb2 = p1ab[None], p1bb[None], p1cb[None]
    p0ab2, p0bb2, p0cb2 = p0ab[None], p0bb[None], p0cb[None]

    pad = EP - E
    srcp = jnp.concatenate([src, jnp.zeros((pad,), jnp.int32)])
    dstp = jnp.concatenate([dst, jnp.full((pad,), N, jnp.int32)])
    src2 = srcp.reshape(EP // K, K)
    dst2 = dstp.reshape(EP // K, K)
    deg2 = _sc_degree(dst2)
    xwp4 = _tc_xwp(x, Wg2, deg2)
    agg4 = _sc_gcn_agg(xwp4, src2, dst2)
    r_o, r_t, score, g = _tc_gcn_post(agg4, xwp4, deg2, bgo2, bgt2,
                                      Wa, ba2)
    m2, pred_t = _tc_msg(score, g, t2, r_t, d1W, d1b2, d2W, d2b2,
                         d3W, d3b2)
    h4 = _sc_attn(m2, src2, dst2)
    z2, pred = _tc_final(h4, r_o, t2, We, be2,
                         (p1aW, p1ab2, p1bW, p1bb2, p1cW, p1cb2),
                         (p0aW, p0ab2, p0bW, p0bb2, p0cW, p0cb2))
    return (pred_t, pred, z2)


# speedup vs baseline: 1.6262x; 1.0028x over previous
import functools

import jax
import jax.numpy as jnp
from jax import lax
from jax.experimental import pallas as pl
from jax.experimental.pallas import tpu as pltpu
from jax.experimental.pallas import tpu_sc as plsc

N = 10000
E = 320000
H = 128
HH = H // 2
NC = 2
NS = 16
K = 128
EP = 323584
NROW = 10240
RPT = NROW // NS

_f32 = jnp.float32


def _mesh():
    return plsc.VectorSubcoreMesh(core_axis_name="c", subcore_axis_name="s")


_SC_PARAMS = pltpu.CompilerParams(use_tc_tiling_on_sc=False)


def _zero_vmem2d(buf, rows, width):
    @pl.loop(0, rows)
    def _(i):
        @pl.loop(0, width, step=16)
        def _(j):
            buf[i, pl.ds(j, 16)] = jnp.zeros((16,), _f32)


def _fill_ones2d(buf, rows, width):
    @pl.loop(0, rows)
    def _(i):
        @pl.loop(0, width, step=16)
        def _(j):
            buf[i, pl.ds(j, 16)] = jnp.ones((16,), _f32)


def _zero_shared_rows(zbuf, zrows, table, row0, nrows):
    full, rem = nrows // zrows, nrows % zrows
    for q in range(full):
        pltpu.sync_copy(zbuf, table.at[pl.ds(row0 + q * zrows, zrows)])
    if rem:
        pltpu.sync_copy(zbuf.at[pl.ds(0, rem)],
                        table.at[pl.ds(row0 + full * zrows, rem)])


def _sc_degree(dst2):
    nchunk = EP // (K * NC * NS)

    @functools.partial(
        pl.kernel,
        out_type=jax.ShapeDtypeStruct((NC, NROW, 16), _f32),
        mesh=_mesh(),
        compiler_params=_SC_PARAMS,
        scratch_types=[
            pltpu.VMEM((nchunk, K), jnp.int32),
            pltpu.VMEM((K, 16), _f32),
            pltpu.VMEM((80, 16), _f32),
            pltpu.VMEM((160, 16), _f32),
            pltpu.VMEM_SHARED((NROW, 16), _f32),
            pltpu.SemaphoreType.DMA,
        ],
    )
    def k(dst_hbm, out_hbm, dst2_v, ones_v, zb_v, stage_v, table_sh, sem):
        c = lax.axis_index("c")
        s = lax.axis_index("s")
        wid = c * NS + s
        _fill_ones2d(ones_v, K, 16)
        _zero_vmem2d(zb_v, 80, 16)
        _zero_shared_rows(zb_v, 80, table_sh, s * RPT, RPT)
        pltpu.sync_copy(dst_hbm.at[pl.ds(wid * nchunk, nchunk)], dst2_v)
        plsc.subcore_barrier()

        @pl.loop(0, nchunk)
        def _(j):
            pltpu.async_copy(ones_v, table_sh.at[dst2_v.at[j]], sem,
                             add=True)

        @pl.loop(0, nchunk)
        def _(j):
            pltpu.make_async_copy(ones_v, table_sh.at[dst2_v.at[0]],
                                  sem).wait()

        plsc.subcore_barrier()
        pltpu.sync_copy(table_sh.at[pl.ds(s * RPT, RPT)],
                        out_hbm.at[c, pl.ds(s * RPT, RPT)])

    return k(dst2)


def _add_offset(idx2_v, nrows, off):
    @pl.loop(0, nrows)
    def _(i):
        @pl.loop(0, K, step=16)
        def _(j):
            idx2_v[i, pl.ds(j, 16)] = idx2_v[i, pl.ds(j, 16)] + off


def _pipe_gather_scatter(nchunk, streams):
    def start_g(j, b):
        for (src_hbm, idx2_v, _, bufs, gsems, _ss, _t) in streams:
            pltpu.async_copy(src_hbm.at[idx2_v.at[j]], bufs[b], gsems[b])

    def wait_g(b):
        for (src_hbm, idx2_v, _, bufs, gsems, _ss, _t) in streams:
            pltpu.make_async_copy(src_hbm.at[idx2_v.at[0]], bufs[b],
                                  gsems[b]).wait()

    def scat(j, b):
        for (_s, _i, dst2_v, bufs, _gs, _ss, table_sh) in streams:
            pltpu.sync_copy(bufs[b], table_sh.at[dst2_v.at[j]], add=True)

    start_g(0, 0)
    start_g(1, 1)

    @pl.loop(0, nchunk, step=2)
    def _(j):
        wait_g(0)
        scat(j, 0)

        @pl.when(j + 2 < nchunk)
        def _():
            start_g(j + 2, 0)

        @pl.when(j + 1 < nchunk)
        def _():
            wait_g(1)
            scat(j + 1, 1)

        @pl.when(j + 3 < nchunk)
        def _():
            start_g(j + 3, 1)


def _sc_gcn_agg(xwp4, src2, dst2):
    nchunk = EP // (K * NS)

    @functools.partial(
        pl.kernel,
        out_type=jax.ShapeDtypeStruct((NC, 2, NROW, HH), _f32),
        mesh=_mesh(),
        compiler_params=_SC_PARAMS,
        scratch_types=[
            pltpu.VMEM((nchunk, K), jnp.int32),
            pltpu.VMEM((nchunk, K), jnp.int32),
            pltpu.VMEM((K, HH), _f32),
            pltpu.VMEM((K, HH), _f32),
            pltpu.VMEM((160, HH), _f32),
            pltpu.VMEM_SHARED((NROW, HH), _f32),
            pltpu.SemaphoreType.DMA,
            pltpu.SemaphoreType.DMA,
            pltpu.SemaphoreType.DMA,
            pltpu.SemaphoreType.DMA,
        ],
    )
    def k(x_hbm, src_hbm, dst_hbm, out_hbm, src2_v, dst2_v, buf0, buf1,
          stage_v, table_sh, sem0, sem1, sem2, sem3):
        c = lax.axis_index("c")
        s = lax.axis_index("s")
        pltpu.sync_copy(src_hbm.at[pl.ds(s * nchunk, nchunk)], src2_v)
        pltpu.sync_copy(dst_hbm.at[pl.ds(s * nchunk, nchunk)], dst2_v)
        _add_offset(src2_v, nchunk, c * (2 * N))
        for q in range(2):
            if q == 1:
                _add_offset(src2_v, nchunk, N)
            _zero_vmem2d(buf0, K, HH)
            _zero_shared_rows(buf0, K, table_sh, s * RPT, RPT)
            plsc.subcore_barrier()
            _pipe_gather_scatter(
                nchunk,
                [(x_hbm, src2_v, dst2_v, (buf0, buf1), (sem0, sem1),
                  (sem2, sem3), table_sh)])
            plsc.subcore_barrier()
            pltpu.sync_copy(table_sh.at[pl.ds(s * RPT, RPT)],
                            out_hbm.at[c, q, pl.ds(s * RPT, RPT)])

    return k(xwp4, src2, dst2)


HA = HH + 16


def _sc_attn(m2, src2, dst2):
    nchunk = EP // (K * NS)

    @functools.partial(
        pl.kernel,
        out_type=jax.ShapeDtypeStruct((NC, NROW, HA), _f32),
        mesh=_mesh(),
        compiler_params=_SC_PARAMS,
        scratch_types=[
            pltpu.VMEM((nchunk, K), jnp.int32),
            pltpu.VMEM((nchunk, K), jnp.int32),
            pltpu.VMEM((K, HA), _f32),
            pltpu.VMEM((K, HA), _f32),
            pltpu.VMEM((160, HA), _f32),
            pltpu.VMEM_SHARED((NROW, HA), _f32),
            pltpu.SemaphoreType.DMA,
            pltpu.SemaphoreType.DMA,
            pltpu.SemaphoreType.DMA,
            pltpu.SemaphoreType.DMA,
        ],
    )
    def k(m_hbm, src_hbm, dst_hbm, out_hbm, src2_v, dst2_v, buf0, buf1,
          stage_v, table_sh, sem0, sem1, sem2, sem3):
        c = lax.axis_index("c")
        s = lax.axis_index("s")
        pltpu.sync_copy(src_hbm.at[pl.ds(s * nchunk, nchunk)], src2_v)
        pltpu.sync_copy(dst_hbm.at[pl.ds(s * nchunk, nchunk)], dst2_v)
        _add_offset(src2_v, nchunk, c * N)
        _zero_vmem2d(buf0, K, HA)
        _zero_shared_rows(buf0, K, table_sh, s * RPT, RPT)
        plsc.subcore_barrier()
        _pipe_gather_scatter(
            nchunk,
            [(m_hbm, src2_v, dst2_v, (buf0, buf1), (sem0, sem1),
              (sem2, sem3), table_sh)])
        plsc.subcore_barrier()
        pltpu.sync_copy(table_sh.at[pl.ds(s * RPT, RPT)],
                        out_hbm.at[c, pl.ds(s * RPT, RPT)])

    return k(m2, src2, dst2)


BN = 1000
_GRID = N // BN
_HI = lax.Precision.DEFAULT


def _dot(a, b):
    return jnp.dot(a, b, preferred_element_type=_f32, precision=_HI)


def _lrelu_(v):
    return jnp.where(v > 0, v, 0.2 * v)


def _mlp3_(v, W1, b1, W2, b2, W3, b3):
    v = _lrelu_(_dot(v, W1) + b1)
    v = _lrelu_(_dot(v, W2) + b2)
    return _dot(v, W3) + b3


def _tc_xwp(x, Wg2, deg2):
    def body(x_ref, w_ref, deg_ref, o_ref):
        q = (pl.program_id(0) // _GRID) % 2
        deg = deg_ref[0, 0, :, 0] + deg_ref[0, 1, :, 0] + 1.0
        dinv = lax.rsqrt(deg)[:, None]
        xw = _dot(x_ref[...], w_ref[0])
        half = jnp.where(q == 0, xw[:, :HH], xw[:, HH:])
        o_ref[...] = half * dinv

    return pl.pallas_call(
        body,
        grid=(4 * _GRID,),
        in_specs=[
            pl.BlockSpec((BN, H), lambda i: (i % _GRID, 0)),
            pl.BlockSpec((1, H, H), lambda i: (i // (2 * _GRID), 0, 0)),
            pl.BlockSpec((1, NC, BN, 16), lambda i: (0, 0, i % _GRID, 0)),
        ],
        out_specs=pl.BlockSpec((BN, HH), lambda i: (i, 0)),
        out_shape=jax.ShapeDtypeStruct((4 * N, HH), _f32),
    )(x, Wg2, deg2[None])


def _tc_gcn_post(agg4, xwp4, deg2, bgo, bgt, Wa, ba):
    def body(agg_ref, xo0_ref, xo1_ref, xt0_ref, xt1_ref, deg_ref,
             bgo_ref, bgt_ref, wao_ref, wat_ref, ba_ref,
             ro_ref, rt_ref, sc_ref, g_ref):
        i = pl.program_id(0)
        deg = deg_ref[0, 0, :, 0] + deg_ref[0, 1, :, 0] + 1.0
        dinv = lax.rsqrt(deg)[:, None]
        agg_o = jnp.concatenate([agg_ref[0, 0, 0], agg_ref[0, 0, 1]], axis=1)
        agg_t = jnp.concatenate([agg_ref[0, 1, 0], agg_ref[0, 1, 1]], axis=1)
        xwp_o = jnp.concatenate([xo0_ref[...], xo1_ref[...]], axis=1)
        xwp_t = jnp.concatenate([xt0_ref[...], xt1_ref[...]], axis=1)
        ro = jnp.maximum((agg_o + xwp_o) * dinv + bgo_ref[...], 0.0)
        rt = jnp.maximum((agg_t + xwp_t) * dinv + bgt_ref[...], 0.0)
        sc = _lrelu_(_dot(ro, wao_ref[...]) + _dot(rt, wat_ref[...])
                     + ba_ref[...])
        ro_ref[...] = ro
        rt_ref[...] = rt
        sc_ref[...] = sc

        @pl.when(i == 0)
        def _():
            g_ref[...] = jnp.full((1, 1), -jnp.inf, _f32)

        g_ref[...] = jnp.maximum(g_ref[...], jnp.max(sc))

    hspec = pl.BlockSpec((BN, HH), lambda i: (i, 0))
    return pl.pallas_call(
        body,
        grid=(_GRID,),
        in_specs=[
            pl.BlockSpec((1, NC, 2, BN, HH), lambda i: (0, 0, 0, i, 0)),
            pl.BlockSpec((BN, HH), lambda i: (i, 0)),
            pl.BlockSpec((BN, HH), lambda i: (_GRID + i, 0)),
            pl.BlockSpec((BN, HH), lambda i: (2 * _GRID + i, 0)),
            pl.BlockSpec((BN, HH), lambda i: (3 * _GRID + i, 0)),
            pl.BlockSpec((1, NC, BN, 16), lambda i: (0, 0, i, 0)),
            pl.BlockSpec((1, H), lambda i: (0, 0)),
            pl.BlockSpec((1, H), lambda i: (0, 0)),
            pl.BlockSpec((H, 1), lambda i: (0, 0)),
            pl.BlockSpec((H, 1), lambda i: (0, 0)),
            pl.BlockSpec((1, 1), lambda i: (0, 0)),
        ],
        out_specs=[
            pl.BlockSpec((BN, H), lambda i: (i, 0)),
            pl.BlockSpec((BN, H), lambda i: (i, 0)),
            pl.BlockSpec((BN, 1), lambda i: (i, 0)),
            pl.BlockSpec((1, 1), lambda i: (0, 0)),
        ],
        out_shape=[
            jax.ShapeDtypeStruct((N, H), _f32),
            jax.ShapeDtypeStruct((N, H), _f32),
            jax.ShapeDtypeStruct((N, 1), _f32),
            jax.ShapeDtypeStruct((1, 1), _f32),
        ],
    )(agg4[None], xwp4, xwp4, xwp4, xwp4, deg2[None], bgo, bgt,
      Wa[:H], Wa[H:], ba)


def _tc_msg(score, g, t2, r_t, d1W, d1b, d2W, d2b, d3W, d3b):
    def body(sc_ref, g_ref, rt_ref, t_ref, w1, b1, w2, b2, w3, b3,
             m_ref, pt_ref):
        q = pl.program_id(0) // _GRID
        ms = jnp.where(t_ref[...] > 0,
                       jnp.exp(sc_ref[...] - g_ref[...]), 0.0)
        rt = rt_ref[...]
        half = jnp.where(q == 0, rt[:, :HH], rt[:, HH:])
        m_ref[...] = jnp.concatenate(
            [ms * half, jnp.broadcast_to(ms, (BN, 16))], axis=1)
        pt_ref[...] = jax.nn.sigmoid(
            _mlp3_(rt, w1[...], b1[...], w2[...], b2[...], w3[...],
                   b3[...]))

    wspec = pl.BlockSpec((H, H), lambda i: (0, 0))
    bspec = pl.BlockSpec((1, H), lambda i: (0, 0))
    return pl.pallas_call(
        body,
        grid=(2 * _GRID,),
        in_specs=[
            pl.BlockSpec((BN, 1), lambda i: (i % _GRID, 0)),
            pl.BlockSpec((1, 1), lambda i: (0, 0)),
            pl.BlockSpec((BN, H), lambda i: (i % _GRID, 0)),
            pl.BlockSpec((BN, 1), lambda i: (i % _GRID, 0)),
            wspec, bspec, wspec, bspec,
            pl.BlockSpec((H, 1), lambda i: (0, 0)),
            pl.BlockSpec((1, 1), lambda i: (0, 0)),
        ],
        out_specs=[
            pl.BlockSpec((BN, HA), lambda i: (i, 0)),
            pl.BlockSpec((BN, 1), lambda i: (i % _GRID, 0)),
        ],
        out_shape=[
            jax.ShapeDtypeStruct((2 * N, HA), _f32),
            jax.ShapeDtypeStruct((N, 1), _f32),
        ],
    )(score, g, r_t, t2, d1W, d1b, d2W, d2b, d3W, d3b)


def _tc_final(h4, r_o, t2, We, be, p1, p0):
    p1aW, p1ab, p1bW, p1bb, p1cW, p1cb = p1
    p0aW, p0ab, p0bW, p0bb, p0cW, p0cb = p0

    def body(h_ref, ro_ref, t_ref, wer, weh, be_ref,
             w1a, b1a, w1b, b1b, w1c, b1c,
             w0a, b0a, w0b, b0b, w0c, b0c, z2_ref, pred_ref):
        ssum = h_ref[0, 0, :, HH]
        rec = (1.0 / (ssum + 1e-9))[:, None]
        h_lo = h_ref[0, 0, :, :HH] * rec
        h_hi = h_ref[0, 1, :, :HH] * rec
        hrow = jnp.concatenate([h_lo, h_hi], axis=1)
        z2 = (_dot(ro_ref[...], wer[...]) + _dot(hrow, weh[...])
              + be_ref[...])
        z2_ref[...] = z2
        pv1 = _mlp3_(z2, w1a[...], b1a[...], w1b[...], b1b[...],
                     w1c[...], b1c[...])
        pv0 = _mlp3_(z2, w0a[...], b0a[...], w0b[...], b0b[...],
                     w0c[...], b0c[...])
        pred_ref[...] = jnp.where(t_ref[...] > 0, pv1, pv0)

    wspec = pl.BlockSpec((H, H), lambda i: (0, 0))
    bspec = pl.BlockSpec((1, H), lambda i: (0, 0))
    cspec = pl.BlockSpec((H, 1), lambda i: (0, 0))
    sspec = pl.BlockSpec((1, 1), lambda i: (0, 0))
    return pl.pallas_call(
        body,
        grid=(_GRID,),
        in_specs=[
            pl.BlockSpec((1, NC, BN, HA), lambda i: (0, 0, i, 0)),
            pl.BlockSpec((BN, H), lambda i: (i, 0)),
            pl.BlockSpec((BN, 1), lambda i: (i, 0)),
            wspec, wspec, bspec,
            wspec, bspec, wspec, bspec, cspec, sspec,
            wspec, bspec, wspec, bspec, cspec, sspec,
        ],
        out_specs=[
            pl.BlockSpec((BN, H), lambda i: (i, 0)),
            pl.BlockSpec((BN, 1), lambda i: (i, 0)),
        ],
        out_shape=[
            jax.ShapeDtypeStruct((N, H), _f32),
            jax.ShapeDtypeStruct((N, 1), _f32),
        ],
    )(h4[None], r_o, t2, We[:H], We[H:], be,
      p1aW, p1ab, p1bW, p1bb, p1cW, p1cb,
      p0aW, p0ab, p0bW, p0bb, p0cW, p0cb)


def kernel(x, t, z, edge_index, Wgo, bgo, Wgt, bgt, Wa, ba, We, be,
           d1W, d1b, d2W, d2b, d3W, d3b, p1aW, p1ab, p1bW, p1bb, p1cW,
           p1cb, p0aW, p0ab, p0bW, p0bb, p0cW, p0cb):
    src = edge_index[0]
    dst = edge_index[1]
    t2 = t[:, None]
    Wg2 = jnp.stack([Wgo, Wgt])
    bgo2, bgt2, be2 = bgo[None], bgt[None], be[None]
    ba2 = ba[None]
    d1b2, d2b2, d3b2 = d1b[None], d2b[None], d3b[None]
    p1ab2, p1bb2, p1cb2 = p1ab[None], p1bb[None], p1cb[None]
    p0ab2, p0bb2, p0cb2 = p0ab[None], p0bb[None], p0cb[None]

    pad = EP - E
    srcp = jnp.concatenate([src, jnp.zeros((pad,), jnp.int32)])
    dstp = jnp.concatenate([dst, jnp.full((pad,), N, jnp.int32)])
    src2 = srcp.reshape(EP // K, K)
    dst2 = dstp.reshape(EP // K, K)
    deg2 = _sc_degree(dst2)
    xwp4 = _tc_xwp(x, Wg2, deg2)
    agg4 = _sc_gcn_agg(xwp4, src2, dst2)
    r_o, r_t, score, g = _tc_gcn_post(agg4, xwp4, deg2, bgo2, bgt2,
                                      Wa, ba2)
    m2, pred_t = _tc_msg(score, g, t2, r_t, d1W, d1b2, d2W, d2b2,
                         d3W, d3b2)
    h4 = _sc_attn(m2, src2, dst2)
    z2, pred = _tc_final(h4, r_o, t2, We, be2,
                         (p1aW, p1ab2, p1bW, p1bb2, p1cW, p1cb2),
                         (p0aW, p0ab2, p0bW, p0bb2, p0cW, p0cb2))
    return (pred_t, pred, z2)
